# Initial kernel scaffold; baseline (speedup 1.0000x reference)
import jax
import jax.numpy as jnp
from jax.experimental import pallas as pl


def kernel(x, edge_index_e0, timestamp_e0, edge_index_e1, timestamp_e1, time_w, time_b, Wp0, bp0, Wp1, bp1, Wg0, al0, ar0, bg0, Wg1, al1, ar1, bg1, W1, b1, W2, b2):
    raise NotImplementedError("write your pallas kernel here")


# speedup vs baseline: 8.8809x; 8.8809x over previous
import functools

import jax
import jax.numpy as jnp
from jax import lax
from jax.experimental import pallas as pl
from jax.experimental.pallas import tpu as pltpu
from jax.experimental.pallas import tpu_sc as plsc

N = 10000
E = 320000
D = 128
TD = 32
H = 128
OUT = 16
FIN = D + TD

NC = 2
NS = 16
NW = NC * NS
EPW = E // NW
SUB = 80
NSUB = EPW // SUB
ROWS2D = E // SUB
RPT = ROWS2D // NW
N_PAD = 10240
STRIPE = N_PAD // NS
ZR = 128
NZC = STRIPE // ZR
ZRB = 32
NZCB = STRIPE // ZRB

f32 = jnp.float32
i32 = jnp.int32


def _ct_body(t0_ref, t1_ref, o_ref):
    o_ref[...] = jnp.maximum(jnp.max(t0_ref[...]),
                             jnp.max(t1_ref[...])).reshape(1, 1)


def _current_time(ts0, ts1):
    return pl.pallas_call(
        _ct_body,
        out_shape=jax.ShapeDtypeStruct((1, 1), f32),
    )(ts0.reshape(625, 512), ts1.reshape(625, 512))


_TB = 3200


def _cos_body(ts0_ref, ts1_ref, ct_ref, w_ref, b_ref, T0_ref, T1_ref):
    ct = ct_ref[0, 0]
    w = w_ref[...]
    b = b_ref[...]
    T0_ref[...] = jnp.cos((ct - ts0_ref[...]) * w + b)
    T1_ref[...] = jnp.cos((ct - ts1_ref[...]) * w + b)


def _time_encodings(ts0, ts1, ct, w, b):
    grid = (E // _TB,)
    return pl.pallas_call(
        _cos_body,
        grid=grid,
        in_specs=[
            pl.BlockSpec((_TB, 1), lambda i: (i, 0)),
            pl.BlockSpec((_TB, 1), lambda i: (i, 0)),
            pl.BlockSpec((1, 1), lambda i: (0, 0)),
            pl.BlockSpec((1, TD), lambda i: (0, 0)),
            pl.BlockSpec((1, TD), lambda i: (0, 0)),
        ],
        out_specs=[
            pl.BlockSpec((_TB, TD), lambda i: (i, 0)),
            pl.BlockSpec((_TB, TD), lambda i: (i, 0)),
        ],
        out_shape=[
            jax.ShapeDtypeStruct((E, TD), f32),
            jax.ShapeDtypeStruct((E, TD), f32),
        ],
    )(ts0.reshape(E, 1), ts1.reshape(E, 1), ct, w.reshape(1, TD),
      b.reshape(1, TD))


_BR = 2000
_NG = N // _BR


def _colsum(mat):
    return jnp.sum(mat, axis=1, keepdims=True)


def _mid_body(s0_ref, c0_ref, s1_ref, c1_ref, x_ref, wp0_ref, bp0_ref,
              wp1_ref, bp1_ref, wg0_ref, wg1_ref, al0_ref, ar0_ref,
              al1_ref, ar1_ref, z0_ref, z1_ref, el0_ref, er0_ref,
              el1_ref, er1_ref, m0_ref, m1_ref, acc_ref):
    i = pl.program_id(0)

    def temporal(s_ref, c_ref, wp_ref, bp_ref):
        s = s_ref[0] + s_ref[1]
        cnt = _colsum(c_ref[...])
        inv = 1.0 / jnp.maximum(cnt, 1.0)
        nz = jnp.where(cnt > 0.0, 1.0, 0.0)
        return jnp.dot(s * inv, wp_ref[...],
                       preferred_element_type=f32) + nz * bp_ref[...]

    t0 = temporal(s0_ref, c0_ref, wp0_ref, bp0_ref)
    t1 = temporal(s1_ref, c1_ref, wp1_ref, bp1_ref)
    xc = jnp.concatenate([x_ref[...], (t0 + t1) * 0.5], axis=1)

    z0 = jnp.dot(xc, wg0_ref[...], preferred_element_type=f32)
    z1 = jnp.dot(xc, wg1_ref[...], preferred_element_type=f32)
    z0_ref[...] = z0
    z1_ref[...] = z1
    el0 = jnp.dot(z0, al0_ref[...], preferred_element_type=f32)
    er0 = jnp.dot(z0, ar0_ref[...], preferred_element_type=f32)
    el1 = jnp.dot(z1, al1_ref[...], preferred_element_type=f32)
    er1 = jnp.dot(z1, ar1_ref[...], preferred_element_type=f32)
    el0_ref[...] = el0
    er0_ref[...] = er0
    el1_ref[...] = el1
    er1_ref[...] = er1

    @pl.when(i == 0)
    def _():
        for k in range(4):
            acc_ref[k] = -jnp.inf

    acc_ref[0] = jnp.maximum(acc_ref[0], jnp.max(el0))
    acc_ref[1] = jnp.maximum(acc_ref[1], jnp.max(er0))
    acc_ref[2] = jnp.maximum(acc_ref[2], jnp.max(el1))
    acc_ref[3] = jnp.maximum(acc_ref[3], jnp.max(er1))

    @pl.when(i == _NG - 1)
    def _():
        s0 = acc_ref[0] + acc_ref[1]
        s1 = acc_ref[2] + acc_ref[3]
        m0_ref[...] = jnp.where(s0 >= 0.0, s0, 0.2 * s0).reshape(1, 1)
        m1_ref[...] = jnp.where(s1 >= 0.0, s1, 0.2 * s1).reshape(1, 1)


def _mid(S0p, c0p, S1p, c1p, x, Wp0, bp0, Wp1, bp1, Wg0, Wg1,
         al0, ar0, al1, ar1):
    grid = (_NG,)
    full = lambda shape: pl.BlockSpec(shape, lambda i: tuple(0 for _ in shape))
    return pl.pallas_call(
        _mid_body,
        grid=grid,
        in_specs=[
            pl.BlockSpec((2, _BR, TD), lambda i: (0, i, 0)),
            pl.BlockSpec((_BR, NW), lambda i: (i, 0)),
            pl.BlockSpec((2, _BR, TD), lambda i: (0, i, 0)),
            pl.BlockSpec((_BR, NW), lambda i: (i, 0)),
            pl.BlockSpec((_BR, D), lambda i: (i, 0)),
            full((TD, TD)), full((1, TD)),
            full((TD, TD)), full((1, TD)),
            full((FIN, H)), full((FIN, H)),
            full((H, 1)), full((H, 1)), full((H, 1)), full((H, 1)),
        ],
        out_specs=[
            pl.BlockSpec((_BR, H), lambda i: (i, 0)),
            pl.BlockSpec((_BR, H), lambda i: (i, 0)),
            pl.BlockSpec((_BR, 1), lambda i: (i, 0)),
            pl.BlockSpec((_BR, 1), lambda i: (i, 0)),
            pl.BlockSpec((_BR, 1), lambda i: (i, 0)),
            pl.BlockSpec((_BR, 1), lambda i: (i, 0)),
            pl.BlockSpec((1, 1), lambda i: (0, 0)),
            pl.BlockSpec((1, 1), lambda i: (0, 0)),
        ],
        out_shape=[
            jax.ShapeDtypeStruct((N, H), f32),
            jax.ShapeDtypeStruct((N, H), f32),
            jax.ShapeDtypeStruct((N, 1), f32),
            jax.ShapeDtypeStruct((N, 1), f32),
            jax.ShapeDtypeStruct((N, 1), f32),
            jax.ShapeDtypeStruct((N, 1), f32),
            jax.ShapeDtypeStruct((1, 1), f32),
            jax.ShapeDtypeStruct((1, 1), f32),
        ],
        scratch_shapes=[pltpu.SMEM((4,), f32)],
    )(S0p, c0p, S1p, c1p, x, Wp0, bp0.reshape(1, TD), Wp1,
      bp1.reshape(1, TD), Wg0, Wg1, al0.reshape(H, 1), ar0.reshape(H, 1),
      al1.reshape(H, 1), ar1.reshape(H, 1))


def _final_body(u0_ref, d0_ref, u1_ref, d1_ref, bg0_ref, bg1_ref,
                w1_ref, b1_ref, w2_ref, b2_ref, o_ref):
    def head(u_ref, d_ref, bg_ref):
        u = u_ref[0] + u_ref[1]
        den = _colsum(d_ref[...])
        safe = jnp.where(den > 0.0, den, 1.0)
        return jnp.where(den > 0.0, u / safe, 0.0) + bg_ref[...]

    h = head(u0_ref, d0_ref, bg0_ref) + head(u1_ref, d1_ref, bg1_ref)
    a1 = jnp.maximum(jnp.dot(h, w1_ref[...], preferred_element_type=f32)
                     + b1_ref[...], 0.0)
    o_ref[...] = jnp.dot(a1, w2_ref[...],
                         preferred_element_type=f32) + b2_ref[...]


def _final(U0p, d0p, U1p, d1p, bg0, bg1, W1, b1, W2p, b2p):
    grid = (_NG,)
    full = lambda shape: pl.BlockSpec(shape, lambda i: tuple(0 for _ in shape))
    return pl.pallas_call(
        _final_body,
        grid=grid,
        in_specs=[
            pl.BlockSpec((2, _BR, H), lambda i: (0, i, 0)),
            pl.BlockSpec((_BR, NW), lambda i: (i, 0)),
            pl.BlockSpec((2, _BR, H), lambda i: (0, i, 0)),
            pl.BlockSpec((_BR, NW), lambda i: (i, 0)),
            full((1, H)), full((1, H)),
            full((H, H)), full((1, H)),
            full((H, H)), full((1, H)),
        ],
        out_specs=pl.BlockSpec((_BR, H), lambda i: (i, 0)),
        out_shape=jax.ShapeDtypeStruct((N, H), f32),
    )(U0p, d0p, U1p, d1p, bg0.reshape(1, H), bg1.reshape(1, H),
      W1, b1.reshape(1, H), W2p, b2p.reshape(1, H))


_ABLK = 2000
_ANB = EPW // _ABLK
_AJ = _ABLK // SUB

_sc_mesh = plsc.VectorSubcoreMesh(core_axis_name="c", subcore_axis_name="s")
_sc_params = pltpu.CompilerParams(needs_layout_passes=False, use_tc_tiling_on_sc=False)


def _sc_time_body(T0, df0, T1, df1, zA,
                  S0p, c0p, S1p, c1p,
                  S_sh, zbuf, obuf, ibuf, Tbuf, dflatbuf, cnt_v, sem):
    c = lax.axis_index("c")
    s = lax.axis_index("s")
    wid = s * NC + c
    ones16 = jnp.full((16,), 1.0, f32)
    zeros16 = jnp.zeros((16,), f32)

    pltpu.sync_copy(zA, zbuf)

    for (Th, dfh, Sp, cp) in ((T0, df0, S0p, c0p),
                              (T1, df1, S1p, c1p)):
        for k in range(NZC):
            pltpu.sync_copy(zbuf, S_sh.at[pl.ds(s * STRIPE + k * ZR, ZR)])

        def zero_cnt(t, _):
            cnt_v[pl.ds(t * 16, 16)] = zeros16
            return ()
        lax.fori_loop(0, N // 16, zero_cnt, ())
        plsc.subcore_barrier()

        for blk in range(_ANB):
            ebase = wid * EPW + blk * _ABLK
            pltpu.sync_copy(dfh.at[pl.ds(ebase, _ABLK)], dflatbuf)
            pltpu.sync_copy(Th.at[pl.ds(ebase, _ABLK)], Tbuf)

            def fire(j, _):
                for g in range(SUB // 16):
                    ibuf[j, pl.ds(g * 16, 16)] = (
                        dflatbuf[pl.ds(j * SUB + g * 16, 16)])
                pltpu.async_copy(Tbuf.at[pl.ds(j * SUB, SUB)],
                                 S_sh.at[ibuf.at[j]], sem, add=True)
                return ()
            lax.fori_loop(0, _AJ, fire, ())

            def cnt_step(g, _):
                didx = dflatbuf[pl.ds(g * 16, 16)]
                plsc.addupdate_scatter(cnt_v, [didx], ones16)
                return ()
            lax.fori_loop(0, _ABLK // 16, cnt_step, ())

            def drain(j, _):
                pltpu.make_async_copy(Tbuf.at[pl.ds(0, SUB)],
                                      S_sh.at[ibuf.at[0]], sem).wait()
                return ()
            lax.fori_loop(0, _AJ, drain, ())
        plsc.subcore_barrier()

        for k in range(NZC):
            r0 = s * STRIPE + k * ZR
            pltpu.sync_copy(S_sh.at[pl.ds(r0, ZR)], obuf)
            pltpu.sync_copy(obuf, Sp.at[c, pl.ds(r0, ZR)])
        pltpu.sync_copy(cnt_v, cp.at[wid, 0])
        plsc.subcore_barrier()


def _sc_time(T0, df0, T1, df1, zA):
    k = pl.kernel(
        _sc_time_body,
        out_type=[
            jax.ShapeDtypeStruct((NC, N_PAD, TD), f32),
            jax.ShapeDtypeStruct((NW, 1, N), f32),
            jax.ShapeDtypeStruct((NC, N_PAD, TD), f32),
            jax.ShapeDtypeStruct((NW, 1, N), f32),
        ],
        mesh=_sc_mesh,
        compiler_params=_sc_params,
        scratch_types=[
            pltpu.VMEM_SHARED((N_PAD, TD), f32),
            pltpu.VMEM((ZR, TD), f32),
            pltpu.VMEM((ZR, TD), f32),
            pltpu.VMEM((_AJ, SUB), i32),
            pltpu.VMEM((_ABLK, TD), f32),
            pltpu.VMEM((_ABLK,), i32),
            pltpu.VMEM((N,), f32),
            pltpu.SemaphoreType.DMA,
        ],
    )
    return k(T0, df0, T1, df1, zA)


def _sc_gat_edge(etype_refs, U_sh, den_v,
                 sbufA, sbufB, dbufA, dbufB, elbA, elbB, erbA, erbB,
                 rows0, rows1, ex0, ex1, mbuf, zbuf, obuf,
                 gsemA, gsemB, c, s, wid):
    (zh, elh, erh, sfh, dfh, Mh, Up, dp) = etype_refs
    zeros16 = jnp.zeros((16,), f32)

    pltpu.sync_copy(Mh, mbuf)

    for k in range(NZCB):
        pltpu.sync_copy(zbuf, U_sh.at[pl.ds(s * STRIPE + k * ZRB, ZRB)])

    def zero_den(t, _):
        den_v[pl.ds(t * 16, 16)] = zeros16
        return ()
    lax.fori_loop(0, N // 16, zero_den, ())
    plsc.subcore_barrier()

    mv = mbuf[...]

    def issue(p, sbuf, dbuf, elb, erb, rows, sem):
        base = wid * EPW + p * SUB
        pltpu.sync_copy(sfh.at[pl.ds(base, SUB)], sbuf)
        pltpu.sync_copy(dfh.at[pl.ds(base, SUB)], dbuf)
        pltpu.async_copy(zh.at[sbuf], rows, sem)
        pltpu.async_copy(elh.at[sbuf], elb, sem)
        pltpu.async_copy(erh.at[dbuf], erb, sem)

    def process(p, sbuf, dbuf, elb, erb, rows, exbuf, sem):
        pltpu.make_async_copy(zh.at[sbuf], rows, sem).wait()
        pltpu.make_async_copy(elh.at[sbuf], elb, sem).wait()
        pltpu.make_async_copy(erh.at[dbuf], erb, sem).wait()
        for g in range(SUB // 16):
            sl = pl.ds(g * 16, 16)
            t = elb[sl] + erb[sl]
            e = jnp.where(t >= 0.0, t, 0.2 * t) - mv
            ex = jnp.exp(e)
            exbuf[sl] = ex
            plsc.addupdate_scatter(den_v, [dbuf[sl]], ex)

        def scale(r, _):
            sp = plsc.load_gather(exbuf, [jnp.full((16,), r, i32)])
            for cc in range(H // 16):
                sl2 = pl.ds(cc * 16, 16)
                rows[r, sl2] = rows[r, sl2] * sp
            return ()
        lax.fori_loop(0, SUB, scale, ())
        pltpu.sync_copy(rows, U_sh.at[dbuf], add=True)

    issue(0, sbufA, dbufA, elbA, erbA, rows0, gsemA)

    def pair(jj, _):
        p0 = 2 * jj
        p1 = p0 + 1

        @pl.when(p1 < NSUB)
        def _():
            issue(p1, sbufB, dbufB, elbB, erbB, rows1, gsemB)
        process(p0, sbufA, dbufA, elbA, erbA, rows0, ex0, gsemA)

        @pl.when(p0 + 2 < NSUB)
        def _():
            issue(p0 + 2, sbufA, dbufA, elbA, erbA, rows0, gsemA)

        @pl.when(p1 < NSUB)
        def _():
            process(p1, sbufB, dbufB, elbB, erbB, rows1, ex1, gsemB)
        return ()

    lax.fori_loop(0, (NSUB + 1) // 2, pair, ())
    plsc.subcore_barrier()

    for k in range(NZCB):
        r0 = s * STRIPE + k * ZRB
        pltpu.sync_copy(U_sh.at[pl.ds(r0, ZRB)], obuf)
        pltpu.sync_copy(obuf, Up.at[c, pl.ds(r0, ZRB)])
    pltpu.sync_copy(den_v, dp.at[wid, 0])
    plsc.subcore_barrier()


def _sc_gat_body(z0, el0, er0, sf0, df0, M0,
                 z1, el1, er1, sf1, df1, M1, zB,
                 U0p, d0p, U1p, d1p,
                 U_sh, den_v, sbufA, sbufB, dbufA, dbufB,
                 elbA, elbB, erbA, erbB,
                 rows0, rows1, ex0, ex1, mbuf, zbuf, obuf, gsemA, gsemB):
    c = lax.axis_index("c")
    s = lax.axis_index("s")
    wid = s * NC + c
    pltpu.sync_copy(zB, zbuf)
    for refs in ((z0, el0, er0, sf0, df0, M0, U0p, d0p),
                 (z1, el1, er1, sf1, df1, M1, U1p, d1p)):
        _sc_gat_edge(refs, U_sh, den_v,
                     sbufA, sbufB, dbufA, dbufB, elbA, elbB, erbA, erbB,
                     rows0, rows1, ex0, ex1, mbuf, zbuf, obuf,
                     gsemA, gsemB, c, s, wid)


def _sc_gat(z0, el0, er0, sf0, df0, M0, z1, el1, er1, sf1, df1, M1, zB):
    k = pl.kernel(
        _sc_gat_body,
        out_type=[
            jax.ShapeDtypeStruct((NC, N_PAD, H), f32),
            jax.ShapeDtypeStruct((NW, 1, N), f32),
            jax.ShapeDtypeStruct((NC, N_PAD, H), f32),
            jax.ShapeDtypeStruct((NW, 1, N), f32),
        ],
        mesh=_sc_mesh,
        compiler_params=_sc_params,
        scratch_types=[
            pltpu.VMEM_SHARED((N_PAD, H), f32),
            pltpu.VMEM((N,), f32),
            pltpu.VMEM((SUB,), i32),
            pltpu.VMEM((SUB,), i32),
            pltpu.VMEM((SUB,), i32),
            pltpu.VMEM((SUB,), i32),
            pltpu.VMEM((SUB,), f32),
            pltpu.VMEM((SUB,), f32),
            pltpu.VMEM((SUB,), f32),
            pltpu.VMEM((SUB,), f32),
            pltpu.VMEM((SUB, H), f32),
            pltpu.VMEM((SUB, H), f32),
            pltpu.VMEM((SUB,), f32),
            pltpu.VMEM((SUB,), f32),
            pltpu.VMEM((16,), f32),
            pltpu.VMEM((ZRB, H), f32),
            pltpu.VMEM((ZRB, H), f32),
            pltpu.SemaphoreType.DMA,
            pltpu.SemaphoreType.DMA,
        ],
    )
    return k(z0, el0, er0, sf0, df0, M0, z1, el1, er1, sf1, df1, M1, zB)


def kernel(x, edge_index_e0, timestamp_e0, edge_index_e1, timestamp_e1,
           time_w, time_b, Wp0, bp0, Wp1, bp1,
           Wg0, al0, ar0, bg0, Wg1, al1, ar1, bg1,
           W1, b1, W2, b2):
    src0 = edge_index_e0[0]
    dst0 = edge_index_e0[1]
    src1 = edge_index_e1[0]
    dst1 = edge_index_e1[1]
    ct = _current_time(timestamp_e0, timestamp_e1)
    T0, T1 = _time_encodings(timestamp_e0, timestamp_e1, ct, time_w, time_b)

    zA = jnp.zeros((ZR, TD), f32)
    S0p, c0p, S1p, c1p = _sc_time(T0, dst0, T1, dst1, zA)

    z0, z1, el0, er0, el1, er1, M0, M1 = _mid(
        S0p[:, :N], c0p.reshape(NW, N).T, S1p[:, :N], c1p.reshape(NW, N).T,
        x, Wp0, bp0, Wp1, bp1, Wg0, Wg1, al0, ar0, al1, ar1)

    zB = jnp.zeros((ZRB, H), f32)
    M0b = jnp.broadcast_to(M0.reshape(1), (16,))
    M1b = jnp.broadcast_to(M1.reshape(1), (16,))
    U0p, d0p, U1p, d1p = _sc_gat(
        z0, el0.reshape(N), er0.reshape(N), src0, dst0, M0b,
        z1, el1.reshape(N), er1.reshape(N), src1, dst1, M1b, zB)

    W2p = jnp.zeros((H, H), f32).at[:, :OUT].set(W2)
    b2p = jnp.zeros((H,), f32).at[:OUT].set(b2)
    logits_pad = _final(U0p[:, :N], d0p.reshape(NW, N).T, U1p[:, :N],
                        d1p.reshape(NW, N).T, bg0, bg1, W1, b1, W2p, b2p)
    return logits_pad[:, :OUT]


# speedup vs baseline: 15.3238x; 1.7255x over previous
import functools

import jax
import jax.numpy as jnp
from jax import lax
from jax.experimental import pallas as pl
from jax.experimental.pallas import tpu as pltpu
from jax.experimental.pallas import tpu_sc as plsc

N = 10000
E = 320000
D = 128
TD = 32
H = 128
OUT = 16
FIN = D + TD

NC = 2
NS = 16
NW = NC * NS
EPW = E // NW
SUB = 80
NSUB = EPW // SUB
ROWS2D = E // SUB
RPT = ROWS2D // NW
N_PAD = 10240
STRIPE = N_PAD // NS
ZR = 128
NZC = STRIPE // ZR
ZRB = 32
NZCB = STRIPE // ZRB

f32 = jnp.float32
i32 = jnp.int32


def _ct_body(t0_ref, t1_ref, o_ref):
    o_ref[...] = jnp.maximum(jnp.max(t0_ref[...]),
                             jnp.max(t1_ref[...])).reshape(1, 1)


def _current_time(ts0, ts1):
    return pl.pallas_call(
        _ct_body,
        out_shape=jax.ShapeDtypeStruct((1, 1), f32),
    )(ts0.reshape(625, 512), ts1.reshape(625, 512))


_EPL = 128 // TD
_TROW = E // _EPL
_TB = 800


def _cos_body(ts0_ref, ts1_ref, ct_ref, w_ref, b_ref, T0_ref, T1_ref):
    ct = ct_ref[0, 0]
    w = w_ref[...]
    b = b_ref[...]
    T0_ref[...] = jnp.cos((ct - ts0_ref[...]) * w + b)
    T1_ref[...] = jnp.cos((ct - ts1_ref[...]) * w + b)


def _time_encodings(ts0, ts1, ct, w, b):
    wt = jnp.tile(w, _EPL).reshape(1, 128)
    bt = jnp.tile(b, _EPL).reshape(1, 128)
    ts0r = jnp.broadcast_to(ts0[:, None], (E, TD)).reshape(_TROW, 128)
    ts1r = jnp.broadcast_to(ts1[:, None], (E, TD)).reshape(_TROW, 128)
    grid = (_TROW // _TB,)
    T0f, T1f = pl.pallas_call(
        _cos_body,
        grid=grid,
        in_specs=[
            pl.BlockSpec((_TB, 128), lambda i: (i, 0)),
            pl.BlockSpec((_TB, 128), lambda i: (i, 0)),
            pl.BlockSpec((1, 1), lambda i: (0, 0)),
            pl.BlockSpec((1, 128), lambda i: (0, 0)),
            pl.BlockSpec((1, 128), lambda i: (0, 0)),
        ],
        out_specs=[
            pl.BlockSpec((_TB, 128), lambda i: (i, 0)),
            pl.BlockSpec((_TB, 128), lambda i: (i, 0)),
        ],
        out_shape=[
            jax.ShapeDtypeStruct((_TROW, 128), f32),
            jax.ShapeDtypeStruct((_TROW, 128), f32),
        ],
    )(ts0r, ts1r, ct, wt, bt)
    return T0f.reshape(E, TD), T1f.reshape(E, TD)


_BR = 2000
_NG = N // _BR


def _colsum(mat):
    return jnp.sum(mat, axis=1, keepdims=True)


def _mid_body(s0_ref, c0_ref, s1_ref, c1_ref, x_ref, wp0_ref, bp0_ref,
              wp1_ref, bp1_ref, wg0_ref, wg1_ref, al0_ref, ar0_ref,
              al1_ref, ar1_ref, z0_ref, z1_ref, el0_ref, er0_ref,
              el1_ref, er1_ref, m0_ref, m1_ref, acc_ref):
    i = pl.program_id(0)

    def temporal(s_ref, c_ref, wp_ref, bp_ref):
        s = s_ref[0] + s_ref[1]
        cnt = _colsum(c_ref[...])
        inv = 1.0 / jnp.maximum(cnt, 1.0)
        nz = jnp.where(cnt > 0.0, 1.0, 0.0)
        return jnp.dot(s * inv, wp_ref[...],
                       preferred_element_type=f32) + nz * bp_ref[...]

    t0 = temporal(s0_ref, c0_ref, wp0_ref, bp0_ref)
    t1 = temporal(s1_ref, c1_ref, wp1_ref, bp1_ref)
    xc = jnp.concatenate([x_ref[...], (t0 + t1) * 0.5], axis=1)

    z0 = jnp.dot(xc, wg0_ref[...], preferred_element_type=f32)
    z1 = jnp.dot(xc, wg1_ref[...], preferred_element_type=f32)
    z0_ref[...] = z0
    z1_ref[...] = z1
    el0 = jnp.dot(z0, al0_ref[...], preferred_element_type=f32)
    er0 = jnp.dot(z0, ar0_ref[...], preferred_element_type=f32)
    el1 = jnp.dot(z1, al1_ref[...], preferred_element_type=f32)
    er1 = jnp.dot(z1, ar1_ref[...], preferred_element_type=f32)
    el0_ref[...] = el0
    er0_ref[...] = er0
    el1_ref[...] = el1
    er1_ref[...] = er1

    @pl.when(i == 0)
    def _():
        for k in range(4):
            acc_ref[k] = -jnp.inf

    acc_ref[0] = jnp.maximum(acc_ref[0], jnp.max(el0))
    acc_ref[1] = jnp.maximum(acc_ref[1], jnp.max(er0))
    acc_ref[2] = jnp.maximum(acc_ref[2], jnp.max(el1))
    acc_ref[3] = jnp.maximum(acc_ref[3], jnp.max(er1))

    @pl.when(i == _NG - 1)
    def _():
        s0 = acc_ref[0] + acc_ref[1]
        s1 = acc_ref[2] + acc_ref[3]
        m0_ref[...] = jnp.where(s0 >= 0.0, s0, 0.2 * s0).reshape(1, 1)
        m1_ref[...] = jnp.where(s1 >= 0.0, s1, 0.2 * s1).reshape(1, 1)


def _mid(S0p, c0p, S1p, c1p, x, Wp0, bp0, Wp1, bp1, Wg0, Wg1,
         al0, ar0, al1, ar1):
    grid = (_NG,)
    full = lambda shape: pl.BlockSpec(shape, lambda i: tuple(0 for _ in shape))
    return pl.pallas_call(
        _mid_body,
        grid=grid,
        in_specs=[
            pl.BlockSpec((2, _BR, TD), lambda i: (0, i, 0)),
            pl.BlockSpec((_BR, NW), lambda i: (i, 0)),
            pl.BlockSpec((2, _BR, TD), lambda i: (0, i, 0)),
            pl.BlockSpec((_BR, NW), lambda i: (i, 0)),
            pl.BlockSpec((_BR, D), lambda i: (i, 0)),
            full((TD, TD)), full((1, TD)),
            full((TD, TD)), full((1, TD)),
            full((FIN, H)), full((FIN, H)),
            full((H, 1)), full((H, 1)), full((H, 1)), full((H, 1)),
        ],
        out_specs=[
            pl.BlockSpec((_BR, H), lambda i: (i, 0)),
            pl.BlockSpec((_BR, H), lambda i: (i, 0)),
            pl.BlockSpec((_BR, 1), lambda i: (i, 0)),
            pl.BlockSpec((_BR, 1), lambda i: (i, 0)),
            pl.BlockSpec((_BR, 1), lambda i: (i, 0)),
            pl.BlockSpec((_BR, 1), lambda i: (i, 0)),
            pl.BlockSpec((1, 1), lambda i: (0, 0)),
            pl.BlockSpec((1, 1), lambda i: (0, 0)),
        ],
        out_shape=[
            jax.ShapeDtypeStruct((N, H), f32),
            jax.ShapeDtypeStruct((N, H), f32),
            jax.ShapeDtypeStruct((N, 1), f32),
            jax.ShapeDtypeStruct((N, 1), f32),
            jax.ShapeDtypeStruct((N, 1), f32),
            jax.ShapeDtypeStruct((N, 1), f32),
            jax.ShapeDtypeStruct((1, 1), f32),
            jax.ShapeDtypeStruct((1, 1), f32),
        ],
        scratch_shapes=[pltpu.SMEM((4,), f32)],
    )(S0p, c0p, S1p, c1p, x, Wp0, bp0.reshape(1, TD), Wp1,
      bp1.reshape(1, TD), Wg0, Wg1, al0.reshape(H, 1), ar0.reshape(H, 1),
      al1.reshape(H, 1), ar1.reshape(H, 1))


def _final_body(u0_ref, d0_ref, u1_ref, d1_ref, bg0_ref, bg1_ref,
                w1_ref, b1_ref, w2_ref, b2_ref, o_ref):
    def head(u_ref, d_ref, bg_ref):
        u = u_ref[0] + u_ref[1]
        den = _colsum(d_ref[...])
        safe = jnp.where(den > 0.0, den, 1.0)
        return jnp.where(den > 0.0, u / safe, 0.0) + bg_ref[...]

    h = head(u0_ref, d0_ref, bg0_ref) + head(u1_ref, d1_ref, bg1_ref)
    a1 = jnp.maximum(jnp.dot(h, w1_ref[...], preferred_element_type=f32)
                     + b1_ref[...], 0.0)
    o_ref[...] = jnp.dot(a1, w2_ref[...],
                         preferred_element_type=f32) + b2_ref[...]


def _final(U0p, d0p, U1p, d1p, bg0, bg1, W1, b1, W2p, b2p):
    grid = (_NG,)
    full = lambda shape: pl.BlockSpec(shape, lambda i: tuple(0 for _ in shape))
    return pl.pallas_call(
        _final_body,
        grid=grid,
        in_specs=[
            pl.BlockSpec((2, _BR, H), lambda i: (0, i, 0)),
            pl.BlockSpec((_BR, NW), lambda i: (i, 0)),
            pl.BlockSpec((2, _BR, H), lambda i: (0, i, 0)),
            pl.BlockSpec((_BR, NW), lambda i: (i, 0)),
            full((1, H)), full((1, H)),
            full((H, H)), full((1, H)),
            full((H, H)), full((1, H)),
        ],
        out_specs=pl.BlockSpec((_BR, H), lambda i: (i, 0)),
        out_shape=jax.ShapeDtypeStruct((N, H), f32),
    )(U0p, d0p, U1p, d1p, bg0.reshape(1, H), bg1.reshape(1, H),
      W1, b1.reshape(1, H), W2p, b2p.reshape(1, H))


_ABLK = 2000
_ANB = EPW // _ABLK
_AJ = _ABLK // SUB

_sc_mesh = plsc.VectorSubcoreMesh(core_axis_name="c", subcore_axis_name="s")
_sc_params = pltpu.CompilerParams(needs_layout_passes=False, use_tc_tiling_on_sc=False)


def _sc_time_body(T0, df0, T1, df1, zA,
                  S0p, c0p, S1p, c1p,
                  S_sh, zbuf, obuf, ibuf, Tbuf, dflatbuf, cnt_v, sem):
    c = lax.axis_index("c")
    s = lax.axis_index("s")
    wid = s * NC + c
    ones16 = jnp.full((16,), 1.0, f32)
    zeros16 = jnp.zeros((16,), f32)

    pltpu.sync_copy(zA, zbuf)

    for (Th, dfh, Sp, cp) in ((T0, df0, S0p, c0p),
                              (T1, df1, S1p, c1p)):
        for k in range(NZC):
            pltpu.sync_copy(zbuf, S_sh.at[pl.ds(s * STRIPE + k * ZR, ZR)])

        def zero_cnt(t, _):
            cnt_v[pl.ds(t * 16, 16)] = zeros16
            return ()
        lax.fori_loop(0, N // 16, zero_cnt, ())
        plsc.subcore_barrier()

        for blk in range(_ANB):
            ebase = wid * EPW + blk * _ABLK
            pltpu.sync_copy(dfh.at[pl.ds(ebase, _ABLK)], dflatbuf)
            pltpu.sync_copy(Th.at[pl.ds(ebase, _ABLK)], Tbuf)

            def fire(j, _):
                for g in range(SUB // 16):
                    ibuf[j, pl.ds(g * 16, 16)] = (
                        dflatbuf[pl.ds(j * SUB + g * 16, 16)])
                pltpu.async_copy(Tbuf.at[pl.ds(j * SUB, SUB)],
                                 S_sh.at[ibuf.at[j]], sem, add=True)
                return ()
            lax.fori_loop(0, _AJ, fire, ())

            def cnt_step(g, _):
                didx = dflatbuf[pl.ds(g * 16, 16)]
                plsc.addupdate_scatter(cnt_v, [didx], ones16)
                return ()
            lax.fori_loop(0, _ABLK // 16, cnt_step, ())

            def drain(j, _):
                pltpu.make_async_copy(Tbuf.at[pl.ds(0, SUB)],
                                      S_sh.at[ibuf.at[0]], sem).wait()
                return ()
            lax.fori_loop(0, _AJ, drain, ())
        plsc.subcore_barrier()

        for k in range(NZC):
            r0 = s * STRIPE + k * ZR
            pltpu.sync_copy(S_sh.at[pl.ds(r0, ZR)], obuf)
            pltpu.sync_copy(obuf, Sp.at[c, pl.ds(r0, ZR)])
        pltpu.sync_copy(cnt_v, cp.at[wid, 0])
        plsc.subcore_barrier()


def _sc_time(T0, df0, T1, df1, zA):
    k = pl.kernel(
        _sc_time_body,
        out_type=[
            jax.ShapeDtypeStruct((NC, N_PAD, TD), f32),
            jax.ShapeDtypeStruct((NW, 1, N), f32),
            jax.ShapeDtypeStruct((NC, N_PAD, TD), f32),
            jax.ShapeDtypeStruct((NW, 1, N), f32),
        ],
        mesh=_sc_mesh,
        compiler_params=_sc_params,
        scratch_types=[
            pltpu.VMEM_SHARED((N_PAD, TD), f32),
            pltpu.VMEM((ZR, TD), f32),
            pltpu.VMEM((ZR, TD), f32),
            pltpu.VMEM((_AJ, SUB), i32),
            pltpu.VMEM((_ABLK, TD), f32),
            pltpu.VMEM((_ABLK,), i32),
            pltpu.VMEM((N,), f32),
            pltpu.SemaphoreType.DMA,
        ],
    )
    return k(T0, df0, T1, df1, zA)


def _sc_gat_edge(etype_refs, U_sh, den_v,
                 sbufA, sbufB, dbufA, dbufB, elbA, elbB, erbA, erbB,
                 rows0, rows1, ex0, ex1, mbuf, zbuf, obuf,
                 gsemA, gsemB, c, s, wid):
    (zh, elh, erh, sfh, dfh, Mh, Up, dp) = etype_refs
    zeros16 = jnp.zeros((16,), f32)

    pltpu.sync_copy(Mh, mbuf)

    for k in range(NZCB):
        pltpu.sync_copy(zbuf, U_sh.at[pl.ds(s * STRIPE + k * ZRB, ZRB)])

    def zero_den(t, _):
        den_v[pl.ds(t * 16, 16)] = zeros16
        return ()
    lax.fori_loop(0, N // 16, zero_den, ())
    plsc.subcore_barrier()

    mv = mbuf[...]

    def issue(p, sbuf, dbuf, elb, erb, rows, sem):
        base = wid * EPW + p * SUB
        pltpu.sync_copy(sfh.at[pl.ds(base, SUB)], sbuf)
        pltpu.sync_copy(dfh.at[pl.ds(base, SUB)], dbuf)
        pltpu.async_copy(zh.at[sbuf], rows, sem)
        pltpu.async_copy(elh.at[sbuf], elb, sem)
        pltpu.async_copy(erh.at[dbuf], erb, sem)

    def process(p, sbuf, dbuf, elb, erb, rows, exbuf, sem):
        pltpu.make_async_copy(zh.at[sbuf], rows, sem).wait()
        pltpu.make_async_copy(elh.at[sbuf], elb, sem).wait()
        pltpu.make_async_copy(erh.at[dbuf], erb, sem).wait()
        for g in range(SUB // 16):
            sl = pl.ds(g * 16, 16)
            t = elb[sl] + erb[sl]
            e = jnp.where(t >= 0.0, t, 0.2 * t) - mv
            ex = jnp.exp(e)
            exbuf[sl] = ex
            plsc.addupdate_scatter(den_v, [dbuf[sl]], ex)

        def scale(r, _):
            sp = plsc.load_gather(exbuf, [jnp.full((16,), r, i32)])
            for cc in range(H // 16):
                sl2 = pl.ds(cc * 16, 16)
                rows[r, sl2] = rows[r, sl2] * sp
            return ()
        lax.fori_loop(0, SUB, scale, ())
        pltpu.sync_copy(rows, U_sh.at[dbuf], add=True)

    issue(0, sbufA, dbufA, elbA, erbA, rows0, gsemA)

    def pair(jj, _):
        p0 = 2 * jj
        p1 = p0 + 1

        @pl.when(p1 < NSUB)
        def _():
            issue(p1, sbufB, dbufB, elbB, erbB, rows1, gsemB)
        process(p0, sbufA, dbufA, elbA, erbA, rows0, ex0, gsemA)

        @pl.when(p0 + 2 < NSUB)
        def _():
            issue(p0 + 2, sbufA, dbufA, elbA, erbA, rows0, gsemA)

        @pl.when(p1 < NSUB)
        def _():
            process(p1, sbufB, dbufB, elbB, erbB, rows1, ex1, gsemB)
        return ()

    lax.fori_loop(0, (NSUB + 1) // 2, pair, ())
    plsc.subcore_barrier()

    for k in range(NZCB):
        r0 = s * STRIPE + k * ZRB
        pltpu.sync_copy(U_sh.at[pl.ds(r0, ZRB)], obuf)
        pltpu.sync_copy(obuf, Up.at[c, pl.ds(r0, ZRB)])
    pltpu.sync_copy(den_v, dp.at[wid, 0])
    plsc.subcore_barrier()


def _sc_gat_body(z0, el0, er0, sf0, df0, M0,
                 z1, el1, er1, sf1, df1, M1, zB,
                 U0p, d0p, U1p, d1p,
                 U_sh, den_v, sbufA, sbufB, dbufA, dbufB,
                 elbA, elbB, erbA, erbB,
                 rows0, rows1, ex0, ex1, mbuf, zbuf, obuf, gsemA, gsemB):
    c = lax.axis_index("c")
    s = lax.axis_index("s")
    wid = s * NC + c
    pltpu.sync_copy(zB, zbuf)
    for refs in ((z0, el0, er0, sf0, df0, M0, U0p, d0p),
                 (z1, el1, er1, sf1, df1, M1, U1p, d1p)):
        _sc_gat_edge(refs, U_sh, den_v,
                     sbufA, sbufB, dbufA, dbufB, elbA, elbB, erbA, erbB,
                     rows0, rows1, ex0, ex1, mbuf, zbuf, obuf,
                     gsemA, gsemB, c, s, wid)


def _sc_gat(z0, el0, er0, sf0, df0, M0, z1, el1, er1, sf1, df1, M1, zB):
    k = pl.kernel(
        _sc_gat_body,
        out_type=[
            jax.ShapeDtypeStruct((NC, N_PAD, H), f32),
            jax.ShapeDtypeStruct((NW, 1, N), f32),
            jax.ShapeDtypeStruct((NC, N_PAD, H), f32),
            jax.ShapeDtypeStruct((NW, 1, N), f32),
        ],
        mesh=_sc_mesh,
        compiler_params=_sc_params,
        scratch_types=[
            pltpu.VMEM_SHARED((N_PAD, H), f32),
            pltpu.VMEM((N,), f32),
            pltpu.VMEM((SUB,), i32),
            pltpu.VMEM((SUB,), i32),
            pltpu.VMEM((SUB,), i32),
            pltpu.VMEM((SUB,), i32),
            pltpu.VMEM((SUB,), f32),
            pltpu.VMEM((SUB,), f32),
            pltpu.VMEM((SUB,), f32),
            pltpu.VMEM((SUB,), f32),
            pltpu.VMEM((SUB, H), f32),
            pltpu.VMEM((SUB, H), f32),
            pltpu.VMEM((SUB,), f32),
            pltpu.VMEM((SUB,), f32),
            pltpu.VMEM((16,), f32),
            pltpu.VMEM((ZRB, H), f32),
            pltpu.VMEM((ZRB, H), f32),
            pltpu.SemaphoreType.DMA,
            pltpu.SemaphoreType.DMA,
        ],
    )
    return k(z0, el0, er0, sf0, df0, M0, z1, el1, er1, sf1, df1, M1, zB)


def kernel(x, edge_index_e0, timestamp_e0, edge_index_e1, timestamp_e1,
           time_w, time_b, Wp0, bp0, Wp1, bp1,
           Wg0, al0, ar0, bg0, Wg1, al1, ar1, bg1,
           W1, b1, W2, b2):
    src0 = edge_index_e0[0]
    dst0 = edge_index_e0[1]
    src1 = edge_index_e1[0]
    dst1 = edge_index_e1[1]
    ct = _current_time(timestamp_e0, timestamp_e1)
    T0, T1 = _time_encodings(timestamp_e0, timestamp_e1, ct, time_w, time_b)

    zA = jnp.zeros((ZR, TD), f32)
    S0p, c0p, S1p, c1p = _sc_time(T0, dst0, T1, dst1, zA)

    z0, z1, el0, er0, el1, er1, M0, M1 = _mid(
        S0p[:, :N], c0p.reshape(NW, N).T, S1p[:, :N], c1p.reshape(NW, N).T,
        x, Wp0, bp0, Wp1, bp1, Wg0, Wg1, al0, ar0, al1, ar1)

    zB = jnp.zeros((ZRB, H), f32)
    M0b = jnp.broadcast_to(M0.reshape(1), (16,))
    M1b = jnp.broadcast_to(M1.reshape(1), (16,))
    U0p, d0p, U1p, d1p = _sc_gat(
        z0, el0.reshape(N), er0.reshape(N), src0, dst0, M0b,
        z1, el1.reshape(N), er1.reshape(N), src1, dst1, M1b, zB)

    W2p = jnp.zeros((H, H), f32).at[:, :OUT].set(W2)
    b2p = jnp.zeros((H,), f32).at[:OUT].set(b2)
    logits_pad = _final(U0p[:, :N], d0p.reshape(NW, N).T, U1p[:, :N],
                        d1p.reshape(NW, N).T, bg0, bg1, W1, b1, W2p, b2p)
    return logits_pad[:, :OUT]


# speedup vs baseline: 18.2388x; 1.1902x over previous
import functools

import jax
import jax.numpy as jnp
from jax import lax
from jax.experimental import pallas as pl
from jax.experimental.pallas import tpu as pltpu
from jax.experimental.pallas import tpu_sc as plsc

N = 10000
E = 320000
D = 128
TD = 32
H = 128
OUT = 16
FIN = D + TD

NC = 2
NS = 16
NW = NC * NS
EPW = E // NW
SUB = 80
NSUB = EPW // SUB
ROWS2D = E // SUB
RPT = ROWS2D // NW
N_PAD = 10240
STRIPE = N_PAD // NS
ZR = 128
NZC = STRIPE // ZR
ZRB = 16
NZCB = STRIPE // ZRB

f32 = jnp.float32
i32 = jnp.int32


def _ct_body(t0_ref, t1_ref, o_ref):
    o_ref[...] = jnp.maximum(jnp.max(t0_ref[...]),
                             jnp.max(t1_ref[...])).reshape(1, 1)


def _current_time(ts0, ts1):
    return pl.pallas_call(
        _ct_body,
        out_shape=jax.ShapeDtypeStruct((1, 1), f32),
    )(ts0.reshape(625, 512), ts1.reshape(625, 512))


_EPL = 128 // TD
_TROW = E // _EPL
_TB = 800


def _cos_body(ts0_ref, ts1_ref, ct_ref, w_ref, b_ref, T0_ref, T1_ref):
    ct = ct_ref[0, 0]
    w = w_ref[...]
    b = b_ref[...]
    T0_ref[...] = jnp.cos((ct - ts0_ref[...]) * w + b)
    T1_ref[...] = jnp.cos((ct - ts1_ref[...]) * w + b)


def _time_encodings(ts0, ts1, ct, w, b):
    wt = jnp.tile(w, _EPL).reshape(1, 128)
    bt = jnp.tile(b, _EPL).reshape(1, 128)
    ts0r = jnp.broadcast_to(ts0[:, None], (E, TD)).reshape(_TROW, 128)
    ts1r = jnp.broadcast_to(ts1[:, None], (E, TD)).reshape(_TROW, 128)
    grid = (_TROW // _TB,)
    T0f, T1f = pl.pallas_call(
        _cos_body,
        grid=grid,
        in_specs=[
            pl.BlockSpec((_TB, 128), lambda i: (i, 0)),
            pl.BlockSpec((_TB, 128), lambda i: (i, 0)),
            pl.BlockSpec((1, 1), lambda i: (0, 0)),
            pl.BlockSpec((1, 128), lambda i: (0, 0)),
            pl.BlockSpec((1, 128), lambda i: (0, 0)),
        ],
        out_specs=[
            pl.BlockSpec((_TB, 128), lambda i: (i, 0)),
            pl.BlockSpec((_TB, 128), lambda i: (i, 0)),
        ],
        out_shape=[
            jax.ShapeDtypeStruct((_TROW, 128), f32),
            jax.ShapeDtypeStruct((_TROW, 128), f32),
        ],
    )(ts0r, ts1r, ct, wt, bt)
    return T0f.reshape(E, TD), T1f.reshape(E, TD)


_BR = 2000
_NG = N // _BR


def _colsum(mat):
    return jnp.sum(mat, axis=1, keepdims=True)


def _mid_body(s0_ref, c0_ref, s1_ref, c1_ref, x_ref, wp0_ref, bp0_ref,
              wp1_ref, bp1_ref, wg0_ref, wg1_ref, al0_ref, ar0_ref,
              al1_ref, ar1_ref, z0_ref, z1_ref, el0_ref, er0_ref,
              el1_ref, er1_ref, m0_ref, m1_ref, acc_ref):
    i = pl.program_id(0)

    def temporal(s_ref, c_ref, wp_ref, bp_ref):
        s = s_ref[0] + s_ref[1]
        cnt = _colsum(c_ref[...])
        inv = 1.0 / jnp.maximum(cnt, 1.0)
        nz = jnp.where(cnt > 0.0, 1.0, 0.0)
        return jnp.dot(s * inv, wp_ref[...],
                       preferred_element_type=f32) + nz * bp_ref[...]

    t0 = temporal(s0_ref, c0_ref, wp0_ref, bp0_ref)
    t1 = temporal(s1_ref, c1_ref, wp1_ref, bp1_ref)
    xc = jnp.concatenate([x_ref[...], (t0 + t1) * 0.5], axis=1)

    z0 = jnp.dot(xc, wg0_ref[...], preferred_element_type=f32)
    z1 = jnp.dot(xc, wg1_ref[...], preferred_element_type=f32)
    z0_ref[...] = z0
    z1_ref[...] = z1
    el0 = jnp.dot(z0, al0_ref[...], preferred_element_type=f32)
    er0 = jnp.dot(z0, ar0_ref[...], preferred_element_type=f32)
    el1 = jnp.dot(z1, al1_ref[...], preferred_element_type=f32)
    er1 = jnp.dot(z1, ar1_ref[...], preferred_element_type=f32)
    el0_ref[...] = el0
    er0_ref[...] = er0
    el1_ref[...] = el1
    er1_ref[...] = er1

    @pl.when(i == 0)
    def _():
        for k in range(4):
            acc_ref[k] = -jnp.inf

    acc_ref[0] = jnp.maximum(acc_ref[0], jnp.max(el0))
    acc_ref[1] = jnp.maximum(acc_ref[1], jnp.max(er0))
    acc_ref[2] = jnp.maximum(acc_ref[2], jnp.max(el1))
    acc_ref[3] = jnp.maximum(acc_ref[3], jnp.max(er1))

    @pl.when(i == _NG - 1)
    def _():
        s0 = acc_ref[0] + acc_ref[1]
        s1 = acc_ref[2] + acc_ref[3]
        m0_ref[...] = jnp.where(s0 >= 0.0, s0, 0.2 * s0).reshape(1, 1)
        m1_ref[...] = jnp.where(s1 >= 0.0, s1, 0.2 * s1).reshape(1, 1)


def _mid(S0p, c0p, S1p, c1p, x, Wp0, bp0, Wp1, bp1, Wg0, Wg1,
         al0, ar0, al1, ar1):
    grid = (_NG,)
    full = lambda shape: pl.BlockSpec(shape, lambda i: tuple(0 for _ in shape))
    return pl.pallas_call(
        _mid_body,
        grid=grid,
        in_specs=[
            pl.BlockSpec((2, _BR, TD), lambda i: (0, i, 0)),
            pl.BlockSpec((_BR, NW), lambda i: (i, 0)),
            pl.BlockSpec((2, _BR, TD), lambda i: (0, i, 0)),
            pl.BlockSpec((_BR, NW), lambda i: (i, 0)),
            pl.BlockSpec((_BR, D), lambda i: (i, 0)),
            full((TD, TD)), full((1, TD)),
            full((TD, TD)), full((1, TD)),
            full((FIN, H)), full((FIN, H)),
            full((H, 1)), full((H, 1)), full((H, 1)), full((H, 1)),
        ],
        out_specs=[
            pl.BlockSpec((_BR, H), lambda i: (i, 0)),
            pl.BlockSpec((_BR, H), lambda i: (i, 0)),
            pl.BlockSpec((_BR, 1), lambda i: (i, 0)),
            pl.BlockSpec((_BR, 1), lambda i: (i, 0)),
            pl.BlockSpec((_BR, 1), lambda i: (i, 0)),
            pl.BlockSpec((_BR, 1), lambda i: (i, 0)),
            pl.BlockSpec((1, 1), lambda i: (0, 0)),
            pl.BlockSpec((1, 1), lambda i: (0, 0)),
        ],
        out_shape=[
            jax.ShapeDtypeStruct((N, H), f32),
            jax.ShapeDtypeStruct((N, H), f32),
            jax.ShapeDtypeStruct((N, 1), f32),
            jax.ShapeDtypeStruct((N, 1), f32),
            jax.ShapeDtypeStruct((N, 1), f32),
            jax.ShapeDtypeStruct((N, 1), f32),
            jax.ShapeDtypeStruct((1, 1), f32),
            jax.ShapeDtypeStruct((1, 1), f32),
        ],
        scratch_shapes=[pltpu.SMEM((4,), f32)],
    )(S0p, c0p, S1p, c1p, x, Wp0, bp0.reshape(1, TD), Wp1,
      bp1.reshape(1, TD), Wg0, Wg1, al0.reshape(H, 1), ar0.reshape(H, 1),
      al1.reshape(H, 1), ar1.reshape(H, 1))


def _final_body(u0_ref, d0_ref, u1_ref, d1_ref, bg0_ref, bg1_ref,
                w1_ref, b1_ref, w2_ref, b2_ref, o_ref):
    def head(u_ref, d_ref, bg_ref):
        u = u_ref[0] + u_ref[1]
        den = _colsum(d_ref[...])
        safe = jnp.where(den > 0.0, den, 1.0)
        return jnp.where(den > 0.0, u / safe, 0.0) + bg_ref[...]

    h = head(u0_ref, d0_ref, bg0_ref) + head(u1_ref, d1_ref, bg1_ref)
    a1 = jnp.maximum(jnp.dot(h, w1_ref[...], preferred_element_type=f32)
                     + b1_ref[...], 0.0)
    o_ref[...] = jnp.dot(a1, w2_ref[...],
                         preferred_element_type=f32) + b2_ref[...]


def _final(U0p, d0p, U1p, d1p, bg0, bg1, W1, b1, W2p, b2p):
    grid = (_NG,)
    full = lambda shape: pl.BlockSpec(shape, lambda i: tuple(0 for _ in shape))
    return pl.pallas_call(
        _final_body,
        grid=grid,
        in_specs=[
            pl.BlockSpec((2, _BR, H), lambda i: (0, i, 0)),
            pl.BlockSpec((_BR, NC), lambda i: (i, 0)),
            pl.BlockSpec((2, _BR, H), lambda i: (0, i, 0)),
            pl.BlockSpec((_BR, NC), lambda i: (i, 0)),
            full((1, H)), full((1, H)),
            full((H, H)), full((1, H)),
            full((H, H)), full((1, H)),
        ],
        out_specs=pl.BlockSpec((_BR, H), lambda i: (i, 0)),
        out_shape=jax.ShapeDtypeStruct((N, H), f32),
    )(U0p, d0p, U1p, d1p, bg0.reshape(1, H), bg1.reshape(1, H),
      W1, b1.reshape(1, H), W2p, b2p.reshape(1, H))


_ABLK = 2000
_ANB = EPW // _ABLK
_AJ = _ABLK // SUB

_sc_mesh = plsc.VectorSubcoreMesh(core_axis_name="c", subcore_axis_name="s")
_sc_params = pltpu.CompilerParams(needs_layout_passes=False, use_tc_tiling_on_sc=False)


def _sc_time_body(T0, df0, T1, df1, zA,
                  S0p, c0p, S1p, c1p,
                  S_sh, zbuf, obuf, ibuf, Tbuf, dflatbuf, cnt_v, sem):
    c = lax.axis_index("c")
    s = lax.axis_index("s")
    wid = s * NC + c
    ones16 = jnp.full((16,), 1.0, f32)
    zeros16 = jnp.zeros((16,), f32)

    pltpu.sync_copy(zA, zbuf)

    for (Th, dfh, Sp, cp) in ((T0, df0, S0p, c0p),
                              (T1, df1, S1p, c1p)):
        for k in range(NZC):
            pltpu.sync_copy(zbuf, S_sh.at[pl.ds(s * STRIPE + k * ZR, ZR)])

        def zero_cnt(t, _):
            cnt_v[pl.ds(t * 16, 16)] = zeros16
            return ()
        lax.fori_loop(0, N // 16, zero_cnt, ())
        plsc.subcore_barrier()

        for blk in range(_ANB):
            ebase = wid * EPW + blk * _ABLK
            pltpu.sync_copy(dfh.at[pl.ds(ebase, _ABLK)], dflatbuf)
            pltpu.sync_copy(Th.at[pl.ds(ebase, _ABLK)], Tbuf)

            def fire(j, _):
                for g in range(SUB // 16):
                    ibuf[j, pl.ds(g * 16, 16)] = (
                        dflatbuf[pl.ds(j * SUB + g * 16, 16)])
                pltpu.async_copy(Tbuf.at[pl.ds(j * SUB, SUB)],
                                 S_sh.at[ibuf.at[j]], sem, add=True)
                return ()
            lax.fori_loop(0, _AJ, fire, ())

            def cnt_step(g, _):
                didx = dflatbuf[pl.ds(g * 16, 16)]
                plsc.addupdate_scatter(cnt_v, [didx], ones16)
                return ()
            lax.fori_loop(0, _ABLK // 16, cnt_step, ())

            def drain(j, _):
                pltpu.make_async_copy(Tbuf.at[pl.ds(0, SUB)],
                                      S_sh.at[ibuf.at[0]], sem).wait()
                return ()
            lax.fori_loop(0, _AJ, drain, ())
        plsc.subcore_barrier()

        for k in range(NZC):
            r0 = s * STRIPE + k * ZR
            pltpu.sync_copy(S_sh.at[pl.ds(r0, ZR)], obuf)
            pltpu.sync_copy(obuf, Sp.at[c, pl.ds(r0, ZR)])
        pltpu.sync_copy(cnt_v, cp.at[wid, 0])
        plsc.subcore_barrier()


def _sc_time(T0, df0, T1, df1, zA):
    k = pl.kernel(
        _sc_time_body,
        out_type=[
            jax.ShapeDtypeStruct((NC, N_PAD, TD), f32),
            jax.ShapeDtypeStruct((NW, 1, N), f32),
            jax.ShapeDtypeStruct((NC, N_PAD, TD), f32),
            jax.ShapeDtypeStruct((NW, 1, N), f32),
        ],
        mesh=_sc_mesh,
        compiler_params=_sc_params,
        scratch_types=[
            pltpu.VMEM_SHARED((N_PAD, TD), f32),
            pltpu.VMEM((ZR, TD), f32),
            pltpu.VMEM((ZR, TD), f32),
            pltpu.VMEM((_AJ, SUB), i32),
            pltpu.VMEM((_ABLK, TD), f32),
            pltpu.VMEM((_ABLK,), i32),
            pltpu.VMEM((N,), f32),
            pltpu.SemaphoreType.DMA,
        ],
    )
    return k(T0, df0, T1, df1, zA)


def _sc_gat_edge(etype_refs, U_sh, den_sh, sfb, dfb,
                 dbufA, dbufB, elbA, elbB, erbA, erbB,
                 rows0, rows1, ex0, ex1, mbuf, zdbuf, zbuf, obuf,
                 gsemA, gsemB, ssemA, ssemB, c, s, wid):
    (zh, elh, erh, sfh, dfh, Mh, Up, dp) = etype_refs
    zeros16 = jnp.zeros((16,), f32)

    pltpu.sync_copy(Mh, mbuf)
    pltpu.sync_copy(sfh.at[pl.ds(wid * EPW, EPW)], sfb)
    pltpu.sync_copy(dfh.at[pl.ds(wid * EPW, EPW)], dfb)

    for k in range(NZCB):
        pltpu.sync_copy(zbuf, U_sh.at[pl.ds(s * STRIPE + k * ZRB, ZRB)])

    def zfill(t, _):
        zdbuf[pl.ds(t * 16, 16)] = zeros16
        return ()
    lax.fori_loop(0, STRIPE // 16, zfill, ())
    pltpu.sync_copy(zdbuf, den_sh.at[pl.ds(s * STRIPE, STRIPE)])
    plsc.subcore_barrier()

    mv = mbuf[...]

    def issue(p, dbuf, elb, erb, rows, sem):
        for g in range(SUB // 16):
            sl = pl.ds(g * 16, 16)
            dbuf[sl] = dfb[pl.ds(p * SUB + g * 16, 16)]
        ssl = sfb.at[pl.ds(p * SUB, SUB)]
        pltpu.async_copy(zh.at[ssl], rows, sem)
        pltpu.async_copy(elh.at[ssl], elb, sem)
        pltpu.async_copy(erh.at[dbuf], erb, sem)

    def process(p, dbuf, elb, erb, rows, exbuf, gsem, ssem):
        ssl = sfb.at[pl.ds(p * SUB, SUB)]
        pltpu.make_async_copy(zh.at[ssl], rows, gsem).wait()
        pltpu.make_async_copy(elh.at[ssl], elb, gsem).wait()
        pltpu.make_async_copy(erh.at[dbuf], erb, gsem).wait()
        for g in range(SUB // 16):
            sl = pl.ds(g * 16, 16)
            t = elb[sl] + erb[sl]
            e = jnp.where(t >= 0.0, t, 0.2 * t) - mv
            exbuf[sl] = jnp.exp(e)
        pltpu.async_copy(exbuf, den_sh.at[dbuf], ssem, add=True)

        def scale(r, _):
            sp = plsc.load_gather(exbuf, [jnp.full((16,), r, i32)])
            for cc in range(H // 16):
                sl2 = pl.ds(cc * 16, 16)
                rows[r, sl2] = rows[r, sl2] * sp
            return ()
        lax.fori_loop(0, SUB, scale, ())
        pltpu.async_copy(rows, U_sh.at[dbuf], ssem, add=True)

    def drain(dbuf, elb, erb, rows, exbuf, ssem):
        pltpu.make_async_copy(exbuf, den_sh.at[dbuf], ssem).wait()
        pltpu.make_async_copy(rows, U_sh.at[dbuf], ssem).wait()

    issue(0, dbufA, elbA, erbA, rows0, gsemA)

    def pair(jj, _):
        p0 = 2 * jj
        p1 = p0 + 1

        @pl.when(jj > 0)
        def _():
            drain(dbufB, elbB, erbB, rows1, ex1, ssemB)

        @pl.when(p1 < NSUB)
        def _():
            issue(p1, dbufB, elbB, erbB, rows1, gsemB)
        process(p0, dbufA, elbA, erbA, rows0, ex0, gsemA, ssemA)

        @pl.when(p0 + 2 < NSUB)
        def _():
            drain(dbufA, elbA, erbA, rows0, ex0, ssemA)
            issue(p0 + 2, dbufA, elbA, erbA, rows0, gsemA)

        @pl.when(p1 < NSUB)
        def _():
            process(p1, dbufB, elbB, erbB, rows1, ex1, gsemB, ssemB)
        return ()

    lax.fori_loop(0, (NSUB + 1) // 2, pair, ())
    drain(dbufA, elbA, erbA, rows0, ex0, ssemA)
    plsc.subcore_barrier()

    for k in range(NZCB):
        r0 = s * STRIPE + k * ZRB
        pltpu.sync_copy(U_sh.at[pl.ds(r0, ZRB)], obuf)
        pltpu.sync_copy(obuf, Up.at[c, pl.ds(r0, ZRB)])
    pltpu.sync_copy(den_sh.at[pl.ds(s * STRIPE, STRIPE)], zdbuf)
    pltpu.sync_copy(zdbuf, dp.at[c, pl.ds(s * STRIPE, STRIPE)])
    plsc.subcore_barrier()


def _sc_gat_body(z0, el0, er0, sf0, df0, M0,
                 z1, el1, er1, sf1, df1, M1, zB,
                 U0p, d0p, U1p, d1p,
                 U_sh, den_sh, sfb, dfb, dbufA, dbufB,
                 elbA, elbB, erbA, erbB,
                 rows0, rows1, ex0, ex1, mbuf, zdbuf, zbuf, obuf,
                 gsemA, gsemB, ssemA, ssemB):
    c = lax.axis_index("c")
    s = lax.axis_index("s")
    wid = s * NC + c
    pltpu.sync_copy(zB, zbuf)
    for refs in ((z0, el0, er0, sf0, df0, M0, U0p, d0p),
                 (z1, el1, er1, sf1, df1, M1, U1p, d1p)):
        _sc_gat_edge(refs, U_sh, den_sh, sfb, dfb,
                     dbufA, dbufB, elbA, elbB, erbA, erbB,
                     rows0, rows1, ex0, ex1, mbuf, zdbuf, zbuf, obuf,
                     gsemA, gsemB, ssemA, ssemB, c, s, wid)


def _sc_gat(z0, el0, er0, sf0, df0, M0, z1, el1, er1, sf1, df1, M1, zB):
    k = pl.kernel(
        _sc_gat_body,
        out_type=[
            jax.ShapeDtypeStruct((NC, N_PAD, H), f32),
            jax.ShapeDtypeStruct((NC, N_PAD), f32),
            jax.ShapeDtypeStruct((NC, N_PAD, H), f32),
            jax.ShapeDtypeStruct((NC, N_PAD), f32),
        ],
        mesh=_sc_mesh,
        compiler_params=_sc_params,
        scratch_types=[
            pltpu.VMEM_SHARED((N_PAD, H), f32),
            pltpu.VMEM_SHARED((N_PAD,), f32),
            pltpu.VMEM((EPW,), i32),
            pltpu.VMEM((EPW,), i32),
            pltpu.VMEM((SUB,), i32),
            pltpu.VMEM((SUB,), i32),
            pltpu.VMEM((SUB,), f32),
            pltpu.VMEM((SUB,), f32),
            pltpu.VMEM((SUB,), f32),
            pltpu.VMEM((SUB,), f32),
            pltpu.VMEM((SUB, H), f32),
            pltpu.VMEM((SUB, H), f32),
            pltpu.VMEM((SUB,), f32),
            pltpu.VMEM((SUB,), f32),
            pltpu.VMEM((16,), f32),
            pltpu.VMEM((STRIPE,), f32),
            pltpu.VMEM((ZRB, H), f32),
            pltpu.VMEM((ZRB, H), f32),
            pltpu.SemaphoreType.DMA,
            pltpu.SemaphoreType.DMA,
            pltpu.SemaphoreType.DMA,
            pltpu.SemaphoreType.DMA,
        ],
    )
    return k(z0, el0, er0, sf0, df0, M0, z1, el1, er1, sf1, df1, M1, zB)


def kernel(x, edge_index_e0, timestamp_e0, edge_index_e1, timestamp_e1,
           time_w, time_b, Wp0, bp0, Wp1, bp1,
           Wg0, al0, ar0, bg0, Wg1, al1, ar1, bg1,
           W1, b1, W2, b2):
    src0 = edge_index_e0[0]
    dst0 = edge_index_e0[1]
    src1 = edge_index_e1[0]
    dst1 = edge_index_e1[1]
    ct = _current_time(timestamp_e0, timestamp_e1)
    T0, T1 = _time_encodings(timestamp_e0, timestamp_e1, ct, time_w, time_b)

    zA = jnp.zeros((ZR, TD), f32)
    S0p, c0p, S1p, c1p = _sc_time(T0, dst0, T1, dst1, zA)

    z0, z1, el0, er0, el1, er1, M0, M1 = _mid(
        S0p[:, :N], c0p.reshape(NW, N).T, S1p[:, :N], c1p.reshape(NW, N).T,
        x, Wp0, bp0, Wp1, bp1, Wg0, Wg1, al0, ar0, al1, ar1)

    zB = jnp.zeros((ZRB, H), f32)
    M0b = jnp.broadcast_to(M0.reshape(1), (16,))
    M1b = jnp.broadcast_to(M1.reshape(1), (16,))
    U0p, d0p, U1p, d1p = _sc_gat(
        z0, el0.reshape(N), er0.reshape(N), src0, dst0, M0b,
        z1, el1.reshape(N), er1.reshape(N), src1, dst1, M1b, zB)

    W2p = jnp.zeros((H, H), f32).at[:, :OUT].set(W2)
    b2p = jnp.zeros((H,), f32).at[:OUT].set(b2)
    logits_pad = _final(U0p[:, :N], d0p[:, :N].T, U1p[:, :N],
                        d1p[:, :N].T, bg0, bg1, W1, b1, W2p, b2p)
    return logits_pad[:, :OUT]


# speedup vs baseline: 19.3207x; 1.0593x over previous
import functools

import jax
import jax.numpy as jnp
from jax import lax
from jax.experimental import pallas as pl
from jax.experimental.pallas import tpu as pltpu
from jax.experimental.pallas import tpu_sc as plsc

N = 10000
E = 320000
D = 128
TD = 32
H = 128
OUT = 16
FIN = D + TD

NC = 2
NS = 16
NW = NC * NS
EPW = E // NW
SUB = 80
NSUB = EPW // SUB
ROWS2D = E // SUB
RPT = ROWS2D // NW
N_PAD = 10240
STRIPE = N_PAD // NS
ZR = 128
NZC = STRIPE // ZR
ZRB = 16
NZCB = STRIPE // ZRB

f32 = jnp.float32
i32 = jnp.int32


def _ct_body(t0_ref, t1_ref, o_ref):
    o_ref[...] = jnp.maximum(jnp.max(t0_ref[...]),
                             jnp.max(t1_ref[...])).reshape(1, 1)


def _current_time(ts0, ts1):
    return pl.pallas_call(
        _ct_body,
        out_shape=jax.ShapeDtypeStruct((1, 1), f32),
    )(ts0.reshape(625, 512), ts1.reshape(625, 512))


_EPL = 128 // TD
_TROW = E // _EPL
_TB = 800

_INV2PI = 0.15915494309189535
_RC1 = 6.28125
_RC2 = 0.0019353071795864769


def _cos_poly(x):
    n = jnp.floor(x * _INV2PI + 0.5)
    y = x - n * _RC1
    y = y - n * _RC2
    u = y * y
    p = jnp.float32(-1.1470745597729725e-11)
    p = p * u + 2.08767569878681e-09
    p = p * u - 2.7557319223985893e-07
    p = p * u + 2.48015873015873e-05
    p = p * u - 1.3888888888888889e-03
    p = p * u + 4.1666666666666664e-02
    p = p * u - 0.5
    return p * u + 1.0


def _cos_body(t00, t01, t02, t03, t10, t11, t12, t13,
              ct_ref, w_ref, b_ref, T0_ref, T1_ref):
    ct = ct_ref[0, 0]
    w = w_ref[...]
    b = b_ref[...]

    def enc(trefs, T_ref):
        d = jnp.concatenate(
            [jnp.broadcast_to(ct - t[...], (_TB, TD)) for t in trefs], axis=1)
        T_ref[...] = _cos_poly(d * w + b)

    enc((t00, t01, t02, t03), T0_ref)
    enc((t10, t11, t12, t13), T1_ref)


def _time_encodings(ts0, ts1, ct, w, b):
    wt = jnp.tile(w, _EPL).reshape(1, 128)
    bt = jnp.tile(b, _EPL).reshape(1, 128)
    t04 = ts0.reshape(_TROW, _EPL)
    t14 = ts1.reshape(_TROW, _EPL)
    cols = [t04[:, k:k + 1] for k in range(_EPL)]
    cols += [t14[:, k:k + 1] for k in range(_EPL)]
    grid = (_TROW // _TB,)
    T0f, T1f = pl.pallas_call(
        _cos_body,
        grid=grid,
        in_specs=(
            [pl.BlockSpec((_TB, 1), lambda i: (i, 0)) for _ in range(8)]
            + [pl.BlockSpec((1, 1), lambda i: (0, 0)),
               pl.BlockSpec((1, 128), lambda i: (0, 0)),
               pl.BlockSpec((1, 128), lambda i: (0, 0))]
        ),
        out_specs=[
            pl.BlockSpec((_TB, 128), lambda i: (i, 0)),
            pl.BlockSpec((_TB, 128), lambda i: (i, 0)),
        ],
        out_shape=[
            jax.ShapeDtypeStruct((_TROW, 128), f32),
            jax.ShapeDtypeStruct((_TROW, 128), f32),
        ],
    )(*cols, ct, wt, bt)
    return T0f.reshape(E, TD), T1f.reshape(E, TD)


_BR = 2000
_NG = N // _BR


def _colsum(mat):
    return jnp.sum(mat, axis=1, keepdims=True)


def _mid_body(s0_ref, c0_ref, s1_ref, c1_ref, x_ref, wp0_ref, bp0_ref,
              wp1_ref, bp1_ref, wg0_ref, wg1_ref, al0_ref, ar0_ref,
              al1_ref, ar1_ref, z0_ref, z1_ref, el0_ref, er0_ref,
              el1_ref, er1_ref, m0_ref, m1_ref, acc_ref):
    i = pl.program_id(0)

    def temporal(s_ref, c_ref, wp_ref, bp_ref):
        s = s_ref[0] + s_ref[1]
        cnt = _colsum(c_ref[...])
        inv = 1.0 / jnp.maximum(cnt, 1.0)
        nz = jnp.where(cnt > 0.0, 1.0, 0.0)
        return jnp.dot(s * inv, wp_ref[...],
                       preferred_element_type=f32) + nz * bp_ref[...]

    t0 = temporal(s0_ref, c0_ref, wp0_ref, bp0_ref)
    t1 = temporal(s1_ref, c1_ref, wp1_ref, bp1_ref)
    xc = jnp.concatenate([x_ref[...], (t0 + t1) * 0.5], axis=1)

    z0 = jnp.dot(xc, wg0_ref[...], preferred_element_type=f32)
    z1 = jnp.dot(xc, wg1_ref[...], preferred_element_type=f32)
    z0_ref[...] = z0
    z1_ref[...] = z1
    el0 = jnp.dot(z0, al0_ref[...], preferred_element_type=f32)
    er0 = jnp.dot(z0, ar0_ref[...], preferred_element_type=f32)
    el1 = jnp.dot(z1, al1_ref[...], preferred_element_type=f32)
    er1 = jnp.dot(z1, ar1_ref[...], preferred_element_type=f32)
    el0_ref[...] = el0
    er0_ref[...] = er0
    el1_ref[...] = el1
    er1_ref[...] = er1

    @pl.when(i == 0)
    def _():
        for k in range(4):
            acc_ref[k] = -jnp.inf

    acc_ref[0] = jnp.maximum(acc_ref[0], jnp.max(el0))
    acc_ref[1] = jnp.maximum(acc_ref[1], jnp.max(er0))
    acc_ref[2] = jnp.maximum(acc_ref[2], jnp.max(el1))
    acc_ref[3] = jnp.maximum(acc_ref[3], jnp.max(er1))

    @pl.when(i == _NG - 1)
    def _():
        s0 = acc_ref[0] + acc_ref[1]
        s1 = acc_ref[2] + acc_ref[3]
        m0_ref[...] = jnp.where(s0 >= 0.0, s0, 0.2 * s0).reshape(1, 1)
        m1_ref[...] = jnp.where(s1 >= 0.0, s1, 0.2 * s1).reshape(1, 1)


def _mid(S0p, c0p, S1p, c1p, x, Wp0, bp0, Wp1, bp1, Wg0, Wg1,
         al0, ar0, al1, ar1):
    grid = (_NG,)
    full = lambda shape: pl.BlockSpec(shape, lambda i: tuple(0 for _ in shape))
    return pl.pallas_call(
        _mid_body,
        grid=grid,
        in_specs=[
            pl.BlockSpec((2, _BR, TD), lambda i: (0, i, 0)),
            pl.BlockSpec((_BR, NW), lambda i: (i, 0)),
            pl.BlockSpec((2, _BR, TD), lambda i: (0, i, 0)),
            pl.BlockSpec((_BR, NW), lambda i: (i, 0)),
            pl.BlockSpec((_BR, D), lambda i: (i, 0)),
            full((TD, TD)), full((1, TD)),
            full((TD, TD)), full((1, TD)),
            full((FIN, H)), full((FIN, H)),
            full((H, 1)), full((H, 1)), full((H, 1)), full((H, 1)),
        ],
        out_specs=[
            pl.BlockSpec((_BR, H), lambda i: (i, 0)),
            pl.BlockSpec((_BR, H), lambda i: (i, 0)),
            pl.BlockSpec((_BR, 1), lambda i: (i, 0)),
            pl.BlockSpec((_BR, 1), lambda i: (i, 0)),
            pl.BlockSpec((_BR, 1), lambda i: (i, 0)),
            pl.BlockSpec((_BR, 1), lambda i: (i, 0)),
            pl.BlockSpec((1, 1), lambda i: (0, 0)),
            pl.BlockSpec((1, 1), lambda i: (0, 0)),
        ],
        out_shape=[
            jax.ShapeDtypeStruct((N, H), f32),
            jax.ShapeDtypeStruct((N, H), f32),
            jax.ShapeDtypeStruct((N, 1), f32),
            jax.ShapeDtypeStruct((N, 1), f32),
            jax.ShapeDtypeStruct((N, 1), f32),
            jax.ShapeDtypeStruct((N, 1), f32),
            jax.ShapeDtypeStruct((1, 1), f32),
            jax.ShapeDtypeStruct((1, 1), f32),
        ],
        scratch_shapes=[pltpu.SMEM((4,), f32)],
    )(S0p, c0p, S1p, c1p, x, Wp0, bp0.reshape(1, TD), Wp1,
      bp1.reshape(1, TD), Wg0, Wg1, al0.reshape(H, 1), ar0.reshape(H, 1),
      al1.reshape(H, 1), ar1.reshape(H, 1))


def _final_body(u0_ref, d0_ref, u1_ref, d1_ref, bg0_ref, bg1_ref,
                w1_ref, b1_ref, w2_ref, b2_ref, o_ref):
    def head(u_ref, d_ref, bg_ref):
        u = u_ref[0] + u_ref[1]
        den = _colsum(d_ref[...])
        safe = jnp.where(den > 0.0, den, 1.0)
        return jnp.where(den > 0.0, u / safe, 0.0) + bg_ref[...]

    h = head(u0_ref, d0_ref, bg0_ref) + head(u1_ref, d1_ref, bg1_ref)
    a1 = jnp.maximum(jnp.dot(h, w1_ref[...], preferred_element_type=f32)
                     + b1_ref[...], 0.0)
    o_ref[...] = jnp.dot(a1, w2_ref[...],
                         preferred_element_type=f32) + b2_ref[...]


def _final(U0p, d0p, U1p, d1p, bg0, bg1, W1, b1, W2p, b2p):
    grid = (_NG,)
    full = lambda shape: pl.BlockSpec(shape, lambda i: tuple(0 for _ in shape))
    return pl.pallas_call(
        _final_body,
        grid=grid,
        in_specs=[
            pl.BlockSpec((2, _BR, H), lambda i: (0, i, 0)),
            pl.BlockSpec((_BR, NC), lambda i: (i, 0)),
            pl.BlockSpec((2, _BR, H), lambda i: (0, i, 0)),
            pl.BlockSpec((_BR, NC), lambda i: (i, 0)),
            full((1, H)), full((1, H)),
            full((H, H)), full((1, H)),
            full((H, H)), full((1, H)),
        ],
        out_specs=pl.BlockSpec((_BR, H), lambda i: (i, 0)),
        out_shape=jax.ShapeDtypeStruct((N, H), f32),
    )(U0p, d0p, U1p, d1p, bg0.reshape(1, H), bg1.reshape(1, H),
      W1, b1.reshape(1, H), W2p, b2p.reshape(1, H))


_ABLK = 2000
_ANB = EPW // _ABLK
_AJ = _ABLK // SUB

_sc_mesh = plsc.VectorSubcoreMesh(core_axis_name="c", subcore_axis_name="s")
_sc_params = pltpu.CompilerParams(needs_layout_passes=False, use_tc_tiling_on_sc=False)


def _sc_time_body(T0, df0, T1, df1, zA,
                  S0p, c0p, S1p, c1p,
                  S_sh, zbuf, obuf, ibuf, Tbuf, dflatbuf, cnt_v, sem):
    c = lax.axis_index("c")
    s = lax.axis_index("s")
    wid = s * NC + c
    ones16 = jnp.full((16,), 1.0, f32)
    zeros16 = jnp.zeros((16,), f32)

    pltpu.sync_copy(zA, zbuf)

    for (Th, dfh, Sp, cp) in ((T0, df0, S0p, c0p),
                              (T1, df1, S1p, c1p)):
        for k in range(NZC):
            pltpu.sync_copy(zbuf, S_sh.at[pl.ds(s * STRIPE + k * ZR, ZR)])

        def zero_cnt(t, _):
            cnt_v[pl.ds(t * 16, 16)] = zeros16
            return ()
        lax.fori_loop(0, N // 16, zero_cnt, ())
        plsc.subcore_barrier()

        for blk in range(_ANB):
            ebase = wid * EPW + blk * _ABLK
            pltpu.sync_copy(dfh.at[pl.ds(ebase, _ABLK)], dflatbuf)
            pltpu.sync_copy(Th.at[pl.ds(ebase, _ABLK)], Tbuf)

            def fire(j, _):
                for g in range(SUB // 16):
                    ibuf[j, pl.ds(g * 16, 16)] = (
                        dflatbuf[pl.ds(j * SUB + g * 16, 16)])
                pltpu.async_copy(Tbuf.at[pl.ds(j * SUB, SUB)],
                                 S_sh.at[ibuf.at[j]], sem, add=True)
                return ()
            lax.fori_loop(0, _AJ, fire, ())

            def cnt_step(g, _):
                didx = dflatbuf[pl.ds(g * 16, 16)]
                plsc.addupdate_scatter(cnt_v, [didx], ones16)
                return ()
            lax.fori_loop(0, _ABLK // 16, cnt_step, ())

            def drain(j, _):
                pltpu.make_async_copy(Tbuf.at[pl.ds(0, SUB)],
                                      S_sh.at[ibuf.at[0]], sem).wait()
                return ()
            lax.fori_loop(0, _AJ, drain, ())
        plsc.subcore_barrier()

        for k in range(NZC):
            r0 = s * STRIPE + k * ZR
            pltpu.sync_copy(S_sh.at[pl.ds(r0, ZR)], obuf)
            pltpu.sync_copy(obuf, Sp.at[c, pl.ds(r0, ZR)])
        pltpu.sync_copy(cnt_v, cp.at[wid, 0])
        plsc.subcore_barrier()


def _sc_time(T0, df0, T1, df1, zA):
    k = pl.kernel(
        _sc_time_body,
        out_type=[
            jax.ShapeDtypeStruct((NC, N_PAD, TD), f32),
            jax.ShapeDtypeStruct((NW, 1, N), f32),
            jax.ShapeDtypeStruct((NC, N_PAD, TD), f32),
            jax.ShapeDtypeStruct((NW, 1, N), f32),
        ],
        mesh=_sc_mesh,
        compiler_params=_sc_params,
        scratch_types=[
            pltpu.VMEM_SHARED((N_PAD, TD), f32),
            pltpu.VMEM((ZR, TD), f32),
            pltpu.VMEM((ZR, TD), f32),
            pltpu.VMEM((_AJ, SUB), i32),
            pltpu.VMEM((_ABLK, TD), f32),
            pltpu.VMEM((_ABLK,), i32),
            pltpu.VMEM((N,), f32),
            pltpu.SemaphoreType.DMA,
        ],
    )
    return k(T0, df0, T1, df1, zA)


def _sc_gat_edge(etype_refs, U_sh, den_sh, sfb, dfb,
                 dbufA, dbufB, elbA, elbB, erbA, erbB,
                 rows0, rows1, ex0, ex1, mbuf, zdbuf, zbuf, obuf,
                 gsemA, gsemB, ssemA, ssemB, c, s, wid):
    (zh, elh, erh, sfh, dfh, Mh, Up, dp) = etype_refs
    zeros16 = jnp.zeros((16,), f32)

    pltpu.sync_copy(Mh, mbuf)
    pltpu.sync_copy(sfh.at[pl.ds(wid * EPW, EPW)], sfb)
    pltpu.sync_copy(dfh.at[pl.ds(wid * EPW, EPW)], dfb)

    for k in range(NZCB):
        pltpu.sync_copy(zbuf, U_sh.at[pl.ds(s * STRIPE + k * ZRB, ZRB)])

    def zfill(t, _):
        zdbuf[pl.ds(t * 16, 16)] = zeros16
        return ()
    lax.fori_loop(0, STRIPE // 16, zfill, ())
    pltpu.sync_copy(zdbuf, den_sh.at[pl.ds(s * STRIPE, STRIPE)])
    plsc.subcore_barrier()

    mv = mbuf[...]

    def issue(p, dbuf, elb, erb, rows, sem):
        for g in range(SUB // 16):
            sl = pl.ds(g * 16, 16)
            dbuf[sl] = dfb[pl.ds(p * SUB + g * 16, 16)]
        ssl = sfb.at[pl.ds(p * SUB, SUB)]
        pltpu.async_copy(zh.at[ssl], rows, sem)
        pltpu.async_copy(elh.at[ssl], elb, sem)
        pltpu.async_copy(erh.at[dbuf], erb, sem)

    def process(p, dbuf, elb, erb, rows, exbuf, gsem, ssem):
        ssl = sfb.at[pl.ds(p * SUB, SUB)]
        pltpu.make_async_copy(zh.at[ssl], rows, gsem).wait()
        pltpu.make_async_copy(elh.at[ssl], elb, gsem).wait()
        pltpu.make_async_copy(erh.at[dbuf], erb, gsem).wait()
        for g in range(SUB // 16):
            sl = pl.ds(g * 16, 16)
            t = elb[sl] + erb[sl]
            e = jnp.where(t >= 0.0, t, 0.2 * t) - mv
            exbuf[sl] = jnp.exp(e)
        pltpu.async_copy(exbuf, den_sh.at[dbuf], ssem, add=True)

        @plsc.parallel_loop(0, SUB, 1, unroll=4)
        def _(r):
            sp = plsc.load_gather(exbuf, [jnp.full((16,), r, i32)])
            for cc in range(H // 16):
                sl2 = pl.ds(cc * 16, 16)
                rows[r, sl2] = rows[r, sl2] * sp
        pltpu.async_copy(rows, U_sh.at[dbuf], ssem, add=True)

    def drain(dbuf, elb, erb, rows, exbuf, ssem):
        pltpu.make_async_copy(exbuf, den_sh.at[dbuf], ssem).wait()
        pltpu.make_async_copy(rows, U_sh.at[dbuf], ssem).wait()

    issue(0, dbufA, elbA, erbA, rows0, gsemA)

    def pair(jj, _):
        p0 = 2 * jj
        p1 = p0 + 1

        @pl.when(jj > 0)
        def _():
            drain(dbufB, elbB, erbB, rows1, ex1, ssemB)

        @pl.when(p1 < NSUB)
        def _():
            issue(p1, dbufB, elbB, erbB, rows1, gsemB)
        process(p0, dbufA, elbA, erbA, rows0, ex0, gsemA, ssemA)

        @pl.when(p0 + 2 < NSUB)
        def _():
            drain(dbufA, elbA, erbA, rows0, ex0, ssemA)
            issue(p0 + 2, dbufA, elbA, erbA, rows0, gsemA)

        @pl.when(p1 < NSUB)
        def _():
            process(p1, dbufB, elbB, erbB, rows1, ex1, gsemB, ssemB)
        return ()

    lax.fori_loop(0, (NSUB + 1) // 2, pair, ())
    drain(dbufA, elbA, erbA, rows0, ex0, ssemA)
    plsc.subcore_barrier()

    for k in range(NZCB):
        r0 = s * STRIPE + k * ZRB
        pltpu.sync_copy(U_sh.at[pl.ds(r0, ZRB)], obuf)
        pltpu.sync_copy(obuf, Up.at[c, pl.ds(r0, ZRB)])
    pltpu.sync_copy(den_sh.at[pl.ds(s * STRIPE, STRIPE)], zdbuf)
    pltpu.sync_copy(zdbuf, dp.at[c, pl.ds(s * STRIPE, STRIPE)])
    plsc.subcore_barrier()


def _sc_gat_body(z0, el0, er0, sf0, df0, M0,
                 z1, el1, er1, sf1, df1, M1, zB,
                 U0p, d0p, U1p, d1p,
                 U_sh, den_sh, sfb, dfb, dbufA, dbufB,
                 elbA, elbB, erbA, erbB,
                 rows0, rows1, ex0, ex1, mbuf, zdbuf, zbuf, obuf,
                 gsemA, gsemB, ssemA, ssemB):
    c = lax.axis_index("c")
    s = lax.axis_index("s")
    wid = s * NC + c
    pltpu.sync_copy(zB, zbuf)
    for refs in ((z0, el0, er0, sf0, df0, M0, U0p, d0p),
                 (z1, el1, er1, sf1, df1, M1, U1p, d1p)):
        _sc_gat_edge(refs, U_sh, den_sh, sfb, dfb,
                     dbufA, dbufB, elbA, elbB, erbA, erbB,
                     rows0, rows1, ex0, ex1, mbuf, zdbuf, zbuf, obuf,
                     gsemA, gsemB, ssemA, ssemB, c, s, wid)


def _sc_gat(z0, el0, er0, sf0, df0, M0, z1, el1, er1, sf1, df1, M1, zB):
    k = pl.kernel(
        _sc_gat_body,
        out_type=[
            jax.ShapeDtypeStruct((NC, N_PAD, H), f32),
            jax.ShapeDtypeStruct((NC, N_PAD), f32),
            jax.ShapeDtypeStruct((NC, N_PAD, H), f32),
            jax.ShapeDtypeStruct((NC, N_PAD), f32),
        ],
        mesh=_sc_mesh,
        compiler_params=_sc_params,
        scratch_types=[
            pltpu.VMEM_SHARED((N_PAD, H), f32),
            pltpu.VMEM_SHARED((N_PAD,), f32),
            pltpu.VMEM((EPW,), i32),
            pltpu.VMEM((EPW,), i32),
            pltpu.VMEM((SUB,), i32),
            pltpu.VMEM((SUB,), i32),
            pltpu.VMEM((SUB,), f32),
            pltpu.VMEM((SUB,), f32),
            pltpu.VMEM((SUB,), f32),
            pltpu.VMEM((SUB,), f32),
            pltpu.VMEM((SUB, H), f32),
            pltpu.VMEM((SUB, H), f32),
            pltpu.VMEM((SUB,), f32),
            pltpu.VMEM((SUB,), f32),
            pltpu.VMEM((16,), f32),
            pltpu.VMEM((STRIPE,), f32),
            pltpu.VMEM((ZRB, H), f32),
            pltpu.VMEM((ZRB, H), f32),
            pltpu.SemaphoreType.DMA,
            pltpu.SemaphoreType.DMA,
            pltpu.SemaphoreType.DMA,
            pltpu.SemaphoreType.DMA,
        ],
    )
    return k(z0, el0, er0, sf0, df0, M0, z1, el1, er1, sf1, df1, M1, zB)


def kernel(x, edge_index_e0, timestamp_e0, edge_index_e1, timestamp_e1,
           time_w, time_b, Wp0, bp0, Wp1, bp1,
           Wg0, al0, ar0, bg0, Wg1, al1, ar1, bg1,
           W1, b1, W2, b2):
    src0 = edge_index_e0[0]
    dst0 = edge_index_e0[1]
    src1 = edge_index_e1[0]
    dst1 = edge_index_e1[1]
    ct = _current_time(timestamp_e0, timestamp_e1)
    T0, T1 = _time_encodings(timestamp_e0, timestamp_e1, ct, time_w, time_b)

    zA = jnp.zeros((ZR, TD), f32)
    S0p, c0p, S1p, c1p = _sc_time(T0, dst0, T1, dst1, zA)

    z0, z1, el0, er0, el1, er1, M0, M1 = _mid(
        S0p[:, :N], c0p.reshape(NW, N).T, S1p[:, :N], c1p.reshape(NW, N).T,
        x, Wp0, bp0, Wp1, bp1, Wg0, Wg1, al0, ar0, al1, ar1)

    zB = jnp.zeros((ZRB, H), f32)
    M0b = jnp.broadcast_to(M0.reshape(1), (16,))
    M1b = jnp.broadcast_to(M1.reshape(1), (16,))
    U0p, d0p, U1p, d1p = _sc_gat(
        z0, el0.reshape(N), er0.reshape(N), src0, dst0, M0b,
        z1, el1.reshape(N), er1.reshape(N), src1, dst1, M1b, zB)

    W2p = jnp.zeros((H, H), f32).at[:, :OUT].set(W2)
    b2p = jnp.zeros((H,), f32).at[:OUT].set(b2)
    logits_pad = _final(U0p[:, :N], d0p[:, :N].T, U1p[:, :N],
                        d1p[:, :N].T, bg0, bg1, W1, b1, W2p, b2p)
    return logits_pad[:, :OUT]


# speedup vs baseline: 26.2573x; 1.3590x over previous
import functools

import jax
import jax.numpy as jnp
from jax import lax
from jax.experimental import pallas as pl
from jax.experimental.pallas import tpu as pltpu
from jax.experimental.pallas import tpu_sc as plsc

N = 10000
E = 320000
D = 128
TD = 32
H = 128
OUT = 16
FIN = D + TD

NC = 2
NS = 16
NW = NC * NS
EPW = E // NW
SUB = 80
NSUB = EPW // SUB
ROWS2D = E // SUB
RPT = ROWS2D // NW
N_PAD = 10240
STRIPE = N_PAD // NS
ZR = 128
NZC = STRIPE // ZR
ZRB = 16
NZCB = STRIPE // ZRB

f32 = jnp.float32
i32 = jnp.int32


def _ct_body(t0_ref, t1_ref, o_ref):
    o_ref[...] = jnp.maximum(jnp.max(t0_ref[...]),
                             jnp.max(t1_ref[...])).reshape(1, 1)


def _current_time(ts0, ts1):
    return pl.pallas_call(
        _ct_body,
        out_shape=jax.ShapeDtypeStruct((1, 1), f32),
    )(ts0.reshape(625, 512), ts1.reshape(625, 512))


_EPL = 128 // TD
_TROW = E // _EPL
_TB = 800

_INV2PI = 0.15915494309189535
_RC1 = 6.28125
_RC2 = 0.0019353071795864769


def _cos_poly(x):
    n = jnp.floor(x * _INV2PI + 0.5)
    y = x - n * _RC1
    y = y - n * _RC2
    u = y * y
    p = jnp.float32(-1.1470745597729725e-11)
    p = p * u + 2.08767569878681e-09
    p = p * u - 2.7557319223985893e-07
    p = p * u + 2.48015873015873e-05
    p = p * u - 1.3888888888888889e-03
    p = p * u + 4.1666666666666664e-02
    p = p * u - 0.5
    return p * u + 1.0


def _cos_body(t04_ref, t14_ref, ct_ref, w_ref, b_ref, T0_ref, T1_ref):
    ct = ct_ref[0, 0]
    w = w_ref[...]
    b = b_ref[...]

    def enc(t4, T_ref):
        d = jnp.concatenate(
            [jnp.broadcast_to(ct - t4[:, k:k + 1], (_TB, TD))
             for k in range(_EPL)], axis=1)
        T_ref[...] = _cos_poly(d * w + b)

    enc(t04_ref[...], T0_ref)
    enc(t14_ref[...], T1_ref)


def _time_encodings(ts0, ts1, ct, w, b):
    wt = jnp.tile(w, _EPL).reshape(1, 128)
    bt = jnp.tile(b, _EPL).reshape(1, 128)
    grid = (_TROW // _TB,)
    T0f, T1f = pl.pallas_call(
        _cos_body,
        grid=grid,
        in_specs=[
            pl.BlockSpec((_TB, _EPL), lambda i: (i, 0)),
            pl.BlockSpec((_TB, _EPL), lambda i: (i, 0)),
            pl.BlockSpec((1, 1), lambda i: (0, 0)),
            pl.BlockSpec((1, 128), lambda i: (0, 0)),
            pl.BlockSpec((1, 128), lambda i: (0, 0)),
        ],
        out_specs=[
            pl.BlockSpec((_TB, 128), lambda i: (i, 0)),
            pl.BlockSpec((_TB, 128), lambda i: (i, 0)),
        ],
        out_shape=[
            jax.ShapeDtypeStruct((_TROW, 128), f32),
            jax.ShapeDtypeStruct((_TROW, 128), f32),
        ],
    )(ts0.reshape(_TROW, _EPL), ts1.reshape(_TROW, _EPL), ct, wt, bt)
    return T0f.reshape(E, TD), T1f.reshape(E, TD)


_BR = 2000
_NG = N // _BR


def _colsum(mat):
    return jnp.sum(mat, axis=1, keepdims=True)


def _mid_body(s0_ref, c0_ref, s1_ref, c1_ref, x_ref, wp0_ref, bp0_ref,
              wp1_ref, bp1_ref, wg0_ref, wg1_ref, al0_ref, ar0_ref,
              al1_ref, ar1_ref, z0_ref, z1_ref, el0_ref, er0_ref,
              el1_ref, er1_ref, m0_ref, m1_ref, acc_ref):
    i = pl.program_id(0)

    def temporal(s_ref, c_ref, wp_ref, bp_ref):
        s = s_ref[0] + s_ref[1]
        cnt = _colsum(c_ref[...])
        inv = 1.0 / jnp.maximum(cnt, 1.0)
        nz = jnp.where(cnt > 0.0, 1.0, 0.0)
        return jnp.dot(s * inv, wp_ref[...],
                       preferred_element_type=f32) + nz * bp_ref[...]

    t0 = temporal(s0_ref, c0_ref, wp0_ref, bp0_ref)
    t1 = temporal(s1_ref, c1_ref, wp1_ref, bp1_ref)
    xc = jnp.concatenate([x_ref[...], (t0 + t1) * 0.5], axis=1)

    z0 = jnp.dot(xc, wg0_ref[...], preferred_element_type=f32)
    z1 = jnp.dot(xc, wg1_ref[...], preferred_element_type=f32)
    z0_ref[...] = z0
    z1_ref[...] = z1
    el0 = jnp.dot(z0, al0_ref[...], preferred_element_type=f32)
    er0 = jnp.dot(z0, ar0_ref[...], preferred_element_type=f32)
    el1 = jnp.dot(z1, al1_ref[...], preferred_element_type=f32)
    er1 = jnp.dot(z1, ar1_ref[...], preferred_element_type=f32)
    el0_ref[...] = el0
    er0_ref[...] = er0
    el1_ref[...] = el1
    er1_ref[...] = er1

    @pl.when(i == 0)
    def _():
        for k in range(4):
            acc_ref[k] = -jnp.inf

    acc_ref[0] = jnp.maximum(acc_ref[0], jnp.max(el0))
    acc_ref[1] = jnp.maximum(acc_ref[1], jnp.max(er0))
    acc_ref[2] = jnp.maximum(acc_ref[2], jnp.max(el1))
    acc_ref[3] = jnp.maximum(acc_ref[3], jnp.max(er1))

    @pl.when(i == _NG - 1)
    def _():
        s0 = acc_ref[0] + acc_ref[1]
        s1 = acc_ref[2] + acc_ref[3]
        m0_ref[...] = jnp.where(s0 >= 0.0, s0, 0.2 * s0).reshape(1, 1)
        m1_ref[...] = jnp.where(s1 >= 0.0, s1, 0.2 * s1).reshape(1, 1)


def _mid(S0p, c0p, S1p, c1p, x, Wp0, bp0, Wp1, bp1, Wg0, Wg1,
         al0, ar0, al1, ar1):
    grid = (_NG,)
    full = lambda shape: pl.BlockSpec(shape, lambda i: tuple(0 for _ in shape))
    return pl.pallas_call(
        _mid_body,
        grid=grid,
        in_specs=[
            pl.BlockSpec((2, _BR, TD), lambda i: (0, i, 0)),
            pl.BlockSpec((_BR, NW), lambda i: (i, 0)),
            pl.BlockSpec((2, _BR, TD), lambda i: (0, i, 0)),
            pl.BlockSpec((_BR, NW), lambda i: (i, 0)),
            pl.BlockSpec((_BR, D), lambda i: (i, 0)),
            full((TD, TD)), full((1, TD)),
            full((TD, TD)), full((1, TD)),
            full((FIN, H)), full((FIN, H)),
            full((H, 1)), full((H, 1)), full((H, 1)), full((H, 1)),
        ],
        out_specs=[
            pl.BlockSpec((_BR, H), lambda i: (i, 0)),
            pl.BlockSpec((_BR, H), lambda i: (i, 0)),
            pl.BlockSpec((_BR, 1), lambda i: (i, 0)),
            pl.BlockSpec((_BR, 1), lambda i: (i, 0)),
            pl.BlockSpec((_BR, 1), lambda i: (i, 0)),
            pl.BlockSpec((_BR, 1), lambda i: (i, 0)),
            pl.BlockSpec((1, 1), lambda i: (0, 0)),
            pl.BlockSpec((1, 1), lambda i: (0, 0)),
        ],
        out_shape=[
            jax.ShapeDtypeStruct((N, H), f32),
            jax.ShapeDtypeStruct((N, H), f32),
            jax.ShapeDtypeStruct((N, 1), f32),
            jax.ShapeDtypeStruct((N, 1), f32),
            jax.ShapeDtypeStruct((N, 1), f32),
            jax.ShapeDtypeStruct((N, 1), f32),
            jax.ShapeDtypeStruct((1, 1), f32),
            jax.ShapeDtypeStruct((1, 1), f32),
        ],
        scratch_shapes=[pltpu.SMEM((4,), f32)],
    )(S0p, c0p, S1p, c1p, x, Wp0, bp0.reshape(1, TD), Wp1,
      bp1.reshape(1, TD), Wg0, Wg1, al0.reshape(H, 1), ar0.reshape(H, 1),
      al1.reshape(H, 1), ar1.reshape(H, 1))


def _final_body(u0_ref, d0_ref, u1_ref, d1_ref, bg0_ref, bg1_ref,
                w1_ref, b1_ref, w2_ref, b2_ref, o_ref):
    def head(u_ref, d_ref, bg_ref):
        u = u_ref[0] + u_ref[1]
        den = _colsum(d_ref[...])
        safe = jnp.where(den > 0.0, den, 1.0)
        return jnp.where(den > 0.0, u / safe, 0.0) + bg_ref[...]

    h = head(u0_ref, d0_ref, bg0_ref) + head(u1_ref, d1_ref, bg1_ref)
    a1 = jnp.maximum(jnp.dot(h, w1_ref[...], preferred_element_type=f32)
                     + b1_ref[...], 0.0)
    o_ref[...] = jnp.dot(a1, w2_ref[...],
                         preferred_element_type=f32) + b2_ref[...]


def _final(U0p, d0p, U1p, d1p, bg0, bg1, W1, b1, W2p, b2p):
    grid = (_NG,)
    full = lambda shape: pl.BlockSpec(shape, lambda i: tuple(0 for _ in shape))
    return pl.pallas_call(
        _final_body,
        grid=grid,
        in_specs=[
            pl.BlockSpec((2, _BR, H), lambda i: (0, i, 0)),
            pl.BlockSpec((_BR, NC), lambda i: (i, 0)),
            pl.BlockSpec((2, _BR, H), lambda i: (0, i, 0)),
            pl.BlockSpec((_BR, NC), lambda i: (i, 0)),
            full((1, H)), full((1, H)),
            full((H, H)), full((1, H)),
            full((H, H)), full((1, H)),
        ],
        out_specs=pl.BlockSpec((_BR, H), lambda i: (i, 0)),
        out_shape=jax.ShapeDtypeStruct((N, H), f32),
    )(U0p, d0p, U1p, d1p, bg0.reshape(1, H), bg1.reshape(1, H),
      W1, b1.reshape(1, H), W2p, b2p.reshape(1, H))


_ABLK = 2000
_ANB = EPW // _ABLK
_AJ = _ABLK // SUB

_sc_mesh = plsc.VectorSubcoreMesh(core_axis_name="c", subcore_axis_name="s")
_sc_params = pltpu.CompilerParams(needs_layout_passes=False, use_tc_tiling_on_sc=False)


def _sc_time_body(T0, df0, T1, df1, zA,
                  S0p, c0p, S1p, c1p,
                  S_sh, zbuf, obuf, ibuf, Tbuf, dflatbuf, cnt_v, sem):
    c = lax.axis_index("c")
    s = lax.axis_index("s")
    wid = s * NC + c
    ones16 = jnp.full((16,), 1.0, f32)
    zeros16 = jnp.zeros((16,), f32)

    pltpu.sync_copy(zA, zbuf)

    for (Th, dfh, Sp, cp) in ((T0, df0, S0p, c0p),
                              (T1, df1, S1p, c1p)):
        for k in range(NZC):
            pltpu.sync_copy(zbuf, S_sh.at[pl.ds(s * STRIPE + k * ZR, ZR)])

        def zero_cnt(t, _):
            cnt_v[pl.ds(t * 16, 16)] = zeros16
            return ()
        lax.fori_loop(0, N // 16, zero_cnt, ())
        plsc.subcore_barrier()

        for blk in range(_ANB):
            ebase = wid * EPW + blk * _ABLK
            pltpu.sync_copy(dfh.at[pl.ds(ebase, _ABLK)], dflatbuf)
            pltpu.sync_copy(Th.at[pl.ds(ebase, _ABLK)], Tbuf)

            def fire(j, _):
                for g in range(SUB // 16):
                    ibuf[j, pl.ds(g * 16, 16)] = (
                        dflatbuf[pl.ds(j * SUB + g * 16, 16)])
                pltpu.async_copy(Tbuf.at[pl.ds(j * SUB, SUB)],
                                 S_sh.at[ibuf.at[j]], sem, add=True)
                return ()
            lax.fori_loop(0, _AJ, fire, ())

            def cnt_step(g, _):
                didx = dflatbuf[pl.ds(g * 16, 16)]
                plsc.addupdate_scatter(cnt_v, [didx], ones16)
                return ()
            lax.fori_loop(0, _ABLK // 16, cnt_step, ())

            def drain(j, _):
                pltpu.make_async_copy(Tbuf.at[pl.ds(0, SUB)],
                                      S_sh.at[ibuf.at[0]], sem).wait()
                return ()
            lax.fori_loop(0, _AJ, drain, ())
        plsc.subcore_barrier()

        for k in range(NZC):
            r0 = s * STRIPE + k * ZR
            pltpu.sync_copy(S_sh.at[pl.ds(r0, ZR)], obuf)
            pltpu.sync_copy(obuf, Sp.at[c, pl.ds(r0, ZR)])
        pltpu.sync_copy(cnt_v, cp.at[wid, 0])
        plsc.subcore_barrier()


def _sc_time(T0, df0, T1, df1, zA):
    k = pl.kernel(
        _sc_time_body,
        out_type=[
            jax.ShapeDtypeStruct((NC, N_PAD, TD), f32),
            jax.ShapeDtypeStruct((NW, 1, N), f32),
            jax.ShapeDtypeStruct((NC, N_PAD, TD), f32),
            jax.ShapeDtypeStruct((NW, 1, N), f32),
        ],
        mesh=_sc_mesh,
        compiler_params=_sc_params,
        scratch_types=[
            pltpu.VMEM_SHARED((N_PAD, TD), f32),
            pltpu.VMEM((ZR, TD), f32),
            pltpu.VMEM((ZR, TD), f32),
            pltpu.VMEM((_AJ, SUB), i32),
            pltpu.VMEM((_ABLK, TD), f32),
            pltpu.VMEM((_ABLK,), i32),
            pltpu.VMEM((N,), f32),
            pltpu.SemaphoreType.DMA,
        ],
    )
    return k(T0, df0, T1, df1, zA)


def _sc_gat_edge(etype_refs, U_sh, den_sh, sfb, dfb,
                 dbufA, dbufB, elbA, elbB, erbA, erbB,
                 rows0, rows1, ex0, ex1, mbuf, zdbuf, zbuf, obuf,
                 gsemA, gsemB, ssemA, ssemB, c, s, wid):
    (zh, elh, erh, sfh, dfh, Mh, Up, dp) = etype_refs
    zeros16 = jnp.zeros((16,), f32)

    pltpu.sync_copy(Mh, mbuf)
    pltpu.sync_copy(sfh.at[pl.ds(wid * EPW, EPW)], sfb)
    pltpu.sync_copy(dfh.at[pl.ds(wid * EPW, EPW)], dfb)

    for k in range(NZCB):
        pltpu.sync_copy(zbuf, U_sh.at[pl.ds(s * STRIPE + k * ZRB, ZRB)])

    def zfill(t, _):
        zdbuf[pl.ds(t * 16, 16)] = zeros16
        return ()
    lax.fori_loop(0, STRIPE // 16, zfill, ())
    pltpu.sync_copy(zdbuf, den_sh.at[pl.ds(s * STRIPE, STRIPE)])
    plsc.subcore_barrier()

    mv = mbuf[...]

    def issue(p, dbuf, elb, erb, rows, sem):
        for g in range(SUB // 16):
            sl = pl.ds(g * 16, 16)
            dbuf[sl] = dfb[pl.ds(p * SUB + g * 16, 16)]
        ssl = sfb.at[pl.ds(p * SUB, SUB)]
        pltpu.async_copy(zh.at[ssl], rows, sem)
        pltpu.async_copy(elh.at[ssl], elb, sem)
        pltpu.async_copy(erh.at[dbuf], erb, sem)

    def process(p, dbuf, elb, erb, rows, exbuf, gsem, ssem):
        ssl = sfb.at[pl.ds(p * SUB, SUB)]
        pltpu.make_async_copy(zh.at[ssl], rows, gsem).wait()
        pltpu.make_async_copy(elh.at[ssl], elb, gsem).wait()
        pltpu.make_async_copy(erh.at[dbuf], erb, gsem).wait()
        for g in range(SUB // 16):
            sl = pl.ds(g * 16, 16)
            t = elb[sl] + erb[sl]
            e = jnp.where(t >= 0.0, t, 0.2 * t) - mv
            exbuf[sl] = jnp.exp(e)
        pltpu.async_copy(exbuf, den_sh.at[dbuf], ssem, add=True)

        @plsc.parallel_loop(0, SUB, 1, unroll=4)
        def _(r):
            sp = plsc.load_gather(exbuf, [jnp.full((16,), r, i32)])
            for cc in range(H // 16):
                sl2 = pl.ds(cc * 16, 16)
                rows[r, sl2] = rows[r, sl2] * sp
        pltpu.async_copy(rows, U_sh.at[dbuf], ssem, add=True)

    def drain(dbuf, elb, erb, rows, exbuf, ssem):
        pltpu.make_async_copy(exbuf, den_sh.at[dbuf], ssem).wait()
        pltpu.make_async_copy(rows, U_sh.at[dbuf], ssem).wait()

    issue(0, dbufA, elbA, erbA, rows0, gsemA)

    def pair(jj, _):
        p0 = 2 * jj
        p1 = p0 + 1

        @pl.when(jj > 0)
        def _():
            drain(dbufB, elbB, erbB, rows1, ex1, ssemB)

        @pl.when(p1 < NSUB)
        def _():
            issue(p1, dbufB, elbB, erbB, rows1, gsemB)
        process(p0, dbufA, elbA, erbA, rows0, ex0, gsemA, ssemA)

        @pl.when(p0 + 2 < NSUB)
        def _():
            drain(dbufA, elbA, erbA, rows0, ex0, ssemA)
            issue(p0 + 2, dbufA, elbA, erbA, rows0, gsemA)

        @pl.when(p1 < NSUB)
        def _():
            process(p1, dbufB, elbB, erbB, rows1, ex1, gsemB, ssemB)
        return ()

    lax.fori_loop(0, (NSUB + 1) // 2, pair, ())
    drain(dbufA, elbA, erbA, rows0, ex0, ssemA)
    plsc.subcore_barrier()

    for k in range(NZCB):
        r0 = s * STRIPE + k * ZRB
        pltpu.sync_copy(U_sh.at[pl.ds(r0, ZRB)], obuf)
        pltpu.sync_copy(obuf, Up.at[c, pl.ds(r0, ZRB)])
    pltpu.sync_copy(den_sh.at[pl.ds(s * STRIPE, STRIPE)], zdbuf)
    pltpu.sync_copy(zdbuf, dp.at[c, pl.ds(s * STRIPE, STRIPE)])
    plsc.subcore_barrier()


def _sc_gat_body(z0, el0, er0, sf0, df0, M0,
                 z1, el1, er1, sf1, df1, M1, zB,
                 U0p, d0p, U1p, d1p,
                 U_sh, den_sh, sfb, dfb, dbufA, dbufB,
                 elbA, elbB, erbA, erbB,
                 rows0, rows1, ex0, ex1, mbuf, zdbuf, zbuf, obuf,
                 gsemA, gsemB, ssemA, ssemB):
    c = lax.axis_index("c")
    s = lax.axis_index("s")
    wid = s * NC + c
    pltpu.sync_copy(zB, zbuf)
    for refs in ((z0, el0, er0, sf0, df0, M0, U0p, d0p),
                 (z1, el1, er1, sf1, df1, M1, U1p, d1p)):
        _sc_gat_edge(refs, U_sh, den_sh, sfb, dfb,
                     dbufA, dbufB, elbA, elbB, erbA, erbB,
                     rows0, rows1, ex0, ex1, mbuf, zdbuf, zbuf, obuf,
                     gsemA, gsemB, ssemA, ssemB, c, s, wid)


def _sc_gat(z0, el0, er0, sf0, df0, M0, z1, el1, er1, sf1, df1, M1, zB):
    k = pl.kernel(
        _sc_gat_body,
        out_type=[
            jax.ShapeDtypeStruct((NC, N_PAD, H), f32),
            jax.ShapeDtypeStruct((NC, N_PAD), f32),
            jax.ShapeDtypeStruct((NC, N_PAD, H), f32),
            jax.ShapeDtypeStruct((NC, N_PAD), f32),
        ],
        mesh=_sc_mesh,
        compiler_params=_sc_params,
        scratch_types=[
            pltpu.VMEM_SHARED((N_PAD, H), f32),
            pltpu.VMEM_SHARED((N_PAD,), f32),
            pltpu.VMEM((EPW,), i32),
            pltpu.VMEM((EPW,), i32),
            pltpu.VMEM((SUB,), i32),
            pltpu.VMEM((SUB,), i32),
            pltpu.VMEM((SUB,), f32),
            pltpu.VMEM((SUB,), f32),
            pltpu.VMEM((SUB,), f32),
            pltpu.VMEM((SUB,), f32),
            pltpu.VMEM((SUB, H), f32),
            pltpu.VMEM((SUB, H), f32),
            pltpu.VMEM((SUB,), f32),
            pltpu.VMEM((SUB,), f32),
            pltpu.VMEM((16,), f32),
            pltpu.VMEM((STRIPE,), f32),
            pltpu.VMEM((ZRB, H), f32),
            pltpu.VMEM((ZRB, H), f32),
            pltpu.SemaphoreType.DMA,
            pltpu.SemaphoreType.DMA,
            pltpu.SemaphoreType.DMA,
            pltpu.SemaphoreType.DMA,
        ],
    )
    return k(z0, el0, er0, sf0, df0, M0, z1, el1, er1, sf1, df1, M1, zB)


def kernel(x, edge_index_e0, timestamp_e0, edge_index_e1, timestamp_e1,
           time_w, time_b, Wp0, bp0, Wp1, bp1,
           Wg0, al0, ar0, bg0, Wg1, al1, ar1, bg1,
           W1, b1, W2, b2):
    src0 = edge_index_e0[0]
    dst0 = edge_index_e0[1]
    src1 = edge_index_e1[0]
    dst1 = edge_index_e1[1]
    ct = _current_time(timestamp_e0, timestamp_e1)
    T0, T1 = _time_encodings(timestamp_e0, timestamp_e1, ct, time_w, time_b)

    zA = jnp.zeros((ZR, TD), f32)
    S0p, c0p, S1p, c1p = _sc_time(T0, dst0, T1, dst1, zA)

    z0, z1, el0, er0, el1, er1, M0, M1 = _mid(
        S0p, c0p.reshape(NW, N).T, S1p, c1p.reshape(NW, N).T,
        x, Wp0, bp0, Wp1, bp1, Wg0, Wg1, al0, ar0, al1, ar1)

    zB = jnp.zeros((ZRB, H), f32)
    M0b = jnp.broadcast_to(M0.reshape(1), (16,))
    M1b = jnp.broadcast_to(M1.reshape(1), (16,))
    U0p, d0p, U1p, d1p = _sc_gat(
        z0, el0.reshape(N), er0.reshape(N), src0, dst0, M0b,
        z1, el1.reshape(N), er1.reshape(N), src1, dst1, M1b, zB)

    W2p = jnp.zeros((H, H), f32).at[:, :OUT].set(W2)
    b2p = jnp.zeros((H,), f32).at[:OUT].set(b2)
    logits_pad = _final(U0p, d0p[:, :N].T, U1p,
                        d1p[:, :N].T, bg0, bg1, W1, b1, W2p, b2p)
    return logits_pad[:, :OUT]


# speedup vs baseline: 26.5730x; 1.0120x over previous
import functools

import jax
import jax.numpy as jnp
from jax import lax
from jax.experimental import pallas as pl
from jax.experimental.pallas import tpu as pltpu
from jax.experimental.pallas import tpu_sc as plsc

N = 10000
E = 320000
D = 128
TD = 32
H = 128
OUT = 16
FIN = D + TD

NC = 2
NS = 16
NW = NC * NS
EPW = E // NW
SUB = 80
NSUB = EPW // SUB
ROWS2D = E // SUB
RPT = ROWS2D // NW
N_PAD = 10240
STRIPE = N_PAD // NS
ZR = 128
NZC = STRIPE // ZR
ZRB = 16
NZCB = STRIPE // ZRB

f32 = jnp.float32
i32 = jnp.int32


def _ct_body(t0_ref, t1_ref, o_ref):
    o_ref[...] = jnp.maximum(jnp.max(t0_ref[...]),
                             jnp.max(t1_ref[...])).reshape(1, 1)


def _current_time(ts0, ts1):
    return pl.pallas_call(
        _ct_body,
        out_shape=jax.ShapeDtypeStruct((1, 1), f32),
    )(ts0.reshape(625, 512), ts1.reshape(625, 512))


_EPL = 128 // TD
_TROW = E // _EPL
_TB = 800

_INV2PI = 0.15915494309189535
_RC1 = 6.28125
_RC2 = 0.0019353071795864769


def _cos_poly(x):
    n = jnp.floor(x * _INV2PI + 0.5)
    y = x - n * _RC1
    y = y - n * _RC2
    u = y * y
    p = jnp.float32(-1.1470745597729725e-11)
    p = p * u + 2.08767569878681e-09
    p = p * u - 2.7557319223985893e-07
    p = p * u + 2.48015873015873e-05
    p = p * u - 1.3888888888888889e-03
    p = p * u + 4.1666666666666664e-02
    p = p * u - 0.5
    return p * u + 1.0


def _cos_body(t04_ref, t14_ref, ct_ref, w_ref, b_ref, T0_ref, T1_ref):
    ct = ct_ref[0, 0]
    w = w_ref[...]
    b = b_ref[...]

    def enc(t4, T_ref):
        d = jnp.concatenate(
            [jnp.broadcast_to(ct - t4[:, k:k + 1], (_TB, TD))
             for k in range(_EPL)], axis=1)
        T_ref[...] = _cos_poly(d * w + b)

    enc(t04_ref[...], T0_ref)
    enc(t14_ref[...], T1_ref)


def _time_encodings(ts0, ts1, ct, w, b):
    wt = jnp.tile(w, _EPL).reshape(1, 128)
    bt = jnp.tile(b, _EPL).reshape(1, 128)
    grid = (_TROW // _TB,)
    T0f, T1f = pl.pallas_call(
        _cos_body,
        grid=grid,
        in_specs=[
            pl.BlockSpec((_TB, _EPL), lambda i: (i, 0)),
            pl.BlockSpec((_TB, _EPL), lambda i: (i, 0)),
            pl.BlockSpec((1, 1), lambda i: (0, 0)),
            pl.BlockSpec((1, 128), lambda i: (0, 0)),
            pl.BlockSpec((1, 128), lambda i: (0, 0)),
        ],
        out_specs=[
            pl.BlockSpec((_TB, 128), lambda i: (i, 0)),
            pl.BlockSpec((_TB, 128), lambda i: (i, 0)),
        ],
        out_shape=[
            jax.ShapeDtypeStruct((_TROW, 128), f32),
            jax.ShapeDtypeStruct((_TROW, 128), f32),
        ],
    )(ts0.reshape(_TROW, _EPL), ts1.reshape(_TROW, _EPL), ct, wt, bt)
    return T0f.reshape(E, TD), T1f.reshape(E, TD)


_BR = 2000
_NG = N // _BR


def _colsum(mat):
    return jnp.sum(mat, axis=1, keepdims=True)


def _mid_body(s0_ref, c0_ref, s1_ref, c1_ref, x_ref, wp0_ref, bp0_ref,
              wp1_ref, bp1_ref, wg0_ref, wg1_ref, al0_ref, ar0_ref,
              al1_ref, ar1_ref, z0_ref, z1_ref, el0_ref, er0_ref,
              el1_ref, er1_ref, m0_ref, m1_ref, acc_ref):
    i = pl.program_id(0)

    def temporal(s_ref, c_ref, wp_ref, bp_ref):
        s = s_ref[0] + s_ref[1]
        cnt = _colsum(c_ref[...])
        inv = 1.0 / jnp.maximum(cnt, 1.0)
        nz = jnp.where(cnt > 0.0, 1.0, 0.0)
        return jnp.dot(s * inv, wp_ref[...],
                       preferred_element_type=f32) + nz * bp_ref[...]

    t0 = temporal(s0_ref, c0_ref, wp0_ref, bp0_ref)
    t1 = temporal(s1_ref, c1_ref, wp1_ref, bp1_ref)
    xc = jnp.concatenate([x_ref[...], (t0 + t1) * 0.5], axis=1)

    z0 = jnp.dot(xc, wg0_ref[...], preferred_element_type=f32)
    z1 = jnp.dot(xc, wg1_ref[...], preferred_element_type=f32)
    z0_ref[...] = z0
    z1_ref[...] = z1
    el0 = jnp.dot(z0, al0_ref[...], preferred_element_type=f32)
    er0 = jnp.dot(z0, ar0_ref[...], preferred_element_type=f32)
    el1 = jnp.dot(z1, al1_ref[...], preferred_element_type=f32)
    er1 = jnp.dot(z1, ar1_ref[...], preferred_element_type=f32)
    el0_ref[...] = el0
    er0_ref[...] = er0
    el1_ref[...] = el1
    er1_ref[...] = er1

    @pl.when(i == 0)
    def _():
        for k in range(4):
            acc_ref[k] = -jnp.inf

    acc_ref[0] = jnp.maximum(acc_ref[0], jnp.max(el0))
    acc_ref[1] = jnp.maximum(acc_ref[1], jnp.max(er0))
    acc_ref[2] = jnp.maximum(acc_ref[2], jnp.max(el1))
    acc_ref[3] = jnp.maximum(acc_ref[3], jnp.max(er1))

    @pl.when(i == _NG - 1)
    def _():
        s0 = acc_ref[0] + acc_ref[1]
        s1 = acc_ref[2] + acc_ref[3]
        m0_ref[...] = jnp.where(s0 >= 0.0, s0, 0.2 * s0).reshape(1, 1)
        m1_ref[...] = jnp.where(s1 >= 0.0, s1, 0.2 * s1).reshape(1, 1)


def _mid(S0p, c0p, S1p, c1p, x, Wp0, bp0, Wp1, bp1, Wg0, Wg1,
         al0, ar0, al1, ar1):
    grid = (_NG,)
    full = lambda shape: pl.BlockSpec(shape, lambda i: tuple(0 for _ in shape))
    return pl.pallas_call(
        _mid_body,
        grid=grid,
        in_specs=[
            pl.BlockSpec((2, _BR, TD), lambda i: (0, i, 0)),
            pl.BlockSpec((_BR, NW), lambda i: (i, 0)),
            pl.BlockSpec((2, _BR, TD), lambda i: (0, i, 0)),
            pl.BlockSpec((_BR, NW), lambda i: (i, 0)),
            pl.BlockSpec((_BR, D), lambda i: (i, 0)),
            full((TD, TD)), full((1, TD)),
            full((TD, TD)), full((1, TD)),
            full((FIN, H)), full((FIN, H)),
            full((H, 1)), full((H, 1)), full((H, 1)), full((H, 1)),
        ],
        out_specs=[
            pl.BlockSpec((_BR, H), lambda i: (i, 0)),
            pl.BlockSpec((_BR, H), lambda i: (i, 0)),
            pl.BlockSpec((_BR, 1), lambda i: (i, 0)),
            pl.BlockSpec((_BR, 1), lambda i: (i, 0)),
            pl.BlockSpec((_BR, 1), lambda i: (i, 0)),
            pl.BlockSpec((_BR, 1), lambda i: (i, 0)),
            pl.BlockSpec((1, 1), lambda i: (0, 0)),
            pl.BlockSpec((1, 1), lambda i: (0, 0)),
        ],
        out_shape=[
            jax.ShapeDtypeStruct((N, H), f32),
            jax.ShapeDtypeStruct((N, H), f32),
            jax.ShapeDtypeStruct((N, 1), f32),
            jax.ShapeDtypeStruct((N, 1), f32),
            jax.ShapeDtypeStruct((N, 1), f32),
            jax.ShapeDtypeStruct((N, 1), f32),
            jax.ShapeDtypeStruct((1, 1), f32),
            jax.ShapeDtypeStruct((1, 1), f32),
        ],
        scratch_shapes=[pltpu.SMEM((4,), f32)],
    )(S0p, c0p, S1p, c1p, x, Wp0, bp0.reshape(1, TD), Wp1,
      bp1.reshape(1, TD), Wg0, Wg1, al0.reshape(H, 1), ar0.reshape(H, 1),
      al1.reshape(H, 1), ar1.reshape(H, 1))


def _final_body(u0_ref, d0_ref, u1_ref, d1_ref, bg0_ref, bg1_ref,
                w1_ref, b1_ref, w2_ref, b2_ref, o_ref):
    def head(u_ref, d_ref, bg_ref):
        u = u_ref[0] + u_ref[1]
        den = _colsum(d_ref[...])
        safe = jnp.where(den > 0.0, den, 1.0)
        return jnp.where(den > 0.0, u / safe, 0.0) + bg_ref[...]

    h = head(u0_ref, d0_ref, bg0_ref) + head(u1_ref, d1_ref, bg1_ref)
    a1 = jnp.maximum(jnp.dot(h, w1_ref[...], preferred_element_type=f32)
                     + b1_ref[...], 0.0)
    o_ref[...] = jnp.dot(a1, w2_ref[...],
                         preferred_element_type=f32) + b2_ref[...]


def _final(U0p, d0p, U1p, d1p, bg0, bg1, W1, b1, W2p, b2p):
    grid = (_NG,)
    full = lambda shape: pl.BlockSpec(shape, lambda i: tuple(0 for _ in shape))
    return pl.pallas_call(
        _final_body,
        grid=grid,
        in_specs=[
            pl.BlockSpec((2, _BR, H), lambda i: (0, i, 0)),
            pl.BlockSpec((_BR, NC), lambda i: (i, 0)),
            pl.BlockSpec((2, _BR, H), lambda i: (0, i, 0)),
            pl.BlockSpec((_BR, NC), lambda i: (i, 0)),
            full((1, H)), full((1, H)),
            full((H, H)), full((1, H)),
            full((H, H)), full((1, H)),
        ],
        out_specs=pl.BlockSpec((_BR, H), lambda i: (i, 0)),
        out_shape=jax.ShapeDtypeStruct((N, H), f32),
    )(U0p, d0p, U1p, d1p, bg0.reshape(1, H), bg1.reshape(1, H),
      W1, b1.reshape(1, H), W2p, b2p.reshape(1, H))


_ABLK = 2000
_ANB = EPW // _ABLK
_AJ = _ABLK // SUB

_sc_mesh = plsc.VectorSubcoreMesh(core_axis_name="c", subcore_axis_name="s")
_sc_params = pltpu.CompilerParams(needs_layout_passes=False,
                                  use_tc_tiling_on_sc=False)
_sc_params_b = pltpu.CompilerParams(needs_layout_passes=False)


def _sc_time_body(T0, df0, T1, df1, zA,
                  S0p, c0p, S1p, c1p,
                  S_sh, zbuf, obuf, ibuf, Tbuf, dflatbuf, cnt_v, sem):
    c = lax.axis_index("c")
    s = lax.axis_index("s")
    wid = s * NC + c
    ones16 = jnp.full((16,), 1.0, f32)
    zeros16 = jnp.zeros((16,), f32)

    pltpu.sync_copy(zA, zbuf)

    for (Th, dfh, Sp, cp) in ((T0, df0, S0p, c0p),
                              (T1, df1, S1p, c1p)):
        for k in range(NZC):
            pltpu.sync_copy(zbuf, S_sh.at[pl.ds(s * STRIPE + k * ZR, ZR)])

        def zero_cnt(t, _):
            cnt_v[pl.ds(t * 16, 16)] = zeros16
            return ()
        lax.fori_loop(0, N // 16, zero_cnt, ())
        plsc.subcore_barrier()

        for blk in range(_ANB):
            ebase = wid * EPW + blk * _ABLK
            pltpu.sync_copy(dfh.at[pl.ds(ebase, _ABLK)], dflatbuf)
            pltpu.sync_copy(Th.at[pl.ds(ebase, _ABLK)], Tbuf)

            def fire(j, _):
                for g in range(SUB // 16):
                    ibuf[j, pl.ds(g * 16, 16)] = (
                        dflatbuf[pl.ds(j * SUB + g * 16, 16)])
                pltpu.async_copy(Tbuf.at[pl.ds(j * SUB, SUB)],
                                 S_sh.at[ibuf.at[j]], sem, add=True)
                return ()
            lax.fori_loop(0, _AJ, fire, ())

            def cnt_step(g, _):
                didx = dflatbuf[pl.ds(g * 16, 16)]
                plsc.addupdate_scatter(cnt_v, [didx], ones16)
                return ()
            lax.fori_loop(0, _ABLK // 16, cnt_step, ())

            def drain(j, _):
                pltpu.make_async_copy(Tbuf.at[pl.ds(0, SUB)],
                                      S_sh.at[ibuf.at[0]], sem).wait()
                return ()
            lax.fori_loop(0, _AJ, drain, ())
        plsc.subcore_barrier()

        for k in range(NZC):
            r0 = s * STRIPE + k * ZR
            pltpu.sync_copy(S_sh.at[pl.ds(r0, ZR)], obuf)
            pltpu.sync_copy(obuf, Sp.at[c, pl.ds(r0, ZR)])
        pltpu.sync_copy(cnt_v, cp.at[wid, 0])
        plsc.subcore_barrier()


def _sc_time(T0, df0, T1, df1, zA):
    k = pl.kernel(
        _sc_time_body,
        out_type=[
            jax.ShapeDtypeStruct((NC, N_PAD, TD), f32),
            jax.ShapeDtypeStruct((NW, 1, N), f32),
            jax.ShapeDtypeStruct((NC, N_PAD, TD), f32),
            jax.ShapeDtypeStruct((NW, 1, N), f32),
        ],
        mesh=_sc_mesh,
        compiler_params=_sc_params,
        scratch_types=[
            pltpu.VMEM_SHARED((N_PAD, TD), f32),
            pltpu.VMEM((ZR, TD), f32),
            pltpu.VMEM((ZR, TD), f32),
            pltpu.VMEM((_AJ, SUB), i32),
            pltpu.VMEM((_ABLK, TD), f32),
            pltpu.VMEM((_ABLK,), i32),
            pltpu.VMEM((N,), f32),
            pltpu.SemaphoreType.DMA,
        ],
    )
    return k(T0, df0, T1, df1, zA)


def _sc_gat_edge(etype_refs, U_sh, den_sh, sfb, dfb,
                 dbufA, dbufB, elbA, elbB, erbA, erbB,
                 rows0, rows1, ex0, ex1, mbuf, zdbuf, zbuf, obuf,
                 gsemA, gsemB, ssemA, ssemB, c, s, wid):
    (zh, elh, erh, sfh, dfh, Mh, Up, dp) = etype_refs
    zeros16 = jnp.zeros((16,), f32)

    pltpu.sync_copy(Mh, mbuf)
    pltpu.sync_copy(sfh.at[pl.ds(wid * EPW, EPW)], sfb)
    pltpu.sync_copy(dfh.at[pl.ds(wid * EPW, EPW)], dfb)

    for k in range(NZCB):
        pltpu.sync_copy(zbuf, U_sh.at[pl.ds(s * STRIPE + k * ZRB, ZRB)])

    def zfill(t, _):
        zdbuf[pl.ds(t * 16, 16)] = zeros16
        return ()
    lax.fori_loop(0, STRIPE // 16, zfill, ())
    pltpu.sync_copy(zdbuf, den_sh.at[pl.ds(s * STRIPE, STRIPE)])
    plsc.subcore_barrier()

    mv = mbuf[...]

    def issue(p, dbuf, elb, erb, rows, sem):
        for g in range(SUB // 16):
            sl = pl.ds(g * 16, 16)
            dbuf[sl] = dfb[pl.ds(p * SUB + g * 16, 16)]
        ssl = sfb.at[pl.ds(p * SUB, SUB)]
        pltpu.async_copy(zh.at[ssl], rows, sem)
        pltpu.async_copy(elh.at[ssl], elb, sem)
        pltpu.async_copy(erh.at[dbuf], erb, sem)

    def process(p, dbuf, elb, erb, rows, exbuf, gsem, ssem):
        ssl = sfb.at[pl.ds(p * SUB, SUB)]
        pltpu.make_async_copy(zh.at[ssl], rows, gsem).wait()
        pltpu.make_async_copy(elh.at[ssl], elb, gsem).wait()
        pltpu.make_async_copy(erh.at[dbuf], erb, gsem).wait()
        for g in range(SUB // 16):
            sl = pl.ds(g * 16, 16)
            t = elb[sl] + erb[sl]
            e = jnp.where(t >= 0.0, t, 0.2 * t) - mv
            exbuf[sl] = jnp.exp(e)
        pltpu.async_copy(exbuf, den_sh.at[dbuf], ssem, add=True)

        @plsc.parallel_loop(0, SUB, 1, unroll=4)
        def _(r):
            sp = plsc.load_gather(exbuf, [jnp.full((16,), r, i32)])
            for cc in range(H // 16):
                sl2 = pl.ds(cc * 16, 16)
                rows[r, sl2] = rows[r, sl2] * sp
        pltpu.async_copy(rows, U_sh.at[dbuf], ssem, add=True)

    def drain(dbuf, elb, erb, rows, exbuf, ssem):
        pltpu.make_async_copy(exbuf, den_sh.at[dbuf], ssem).wait()
        pltpu.make_async_copy(rows, U_sh.at[dbuf], ssem).wait()

    issue(0, dbufA, elbA, erbA, rows0, gsemA)

    def pair(jj, _):
        p0 = 2 * jj
        p1 = p0 + 1

        @pl.when(jj > 0)
        def _():
            drain(dbufB, elbB, erbB, rows1, ex1, ssemB)

        @pl.when(p1 < NSUB)
        def _():
            issue(p1, dbufB, elbB, erbB, rows1, gsemB)
        process(p0, dbufA, elbA, erbA, rows0, ex0, gsemA, ssemA)

        @pl.when(p0 + 2 < NSUB)
        def _():
            drain(dbufA, elbA, erbA, rows0, ex0, ssemA)
            issue(p0 + 2, dbufA, elbA, erbA, rows0, gsemA)

        @pl.when(p1 < NSUB)
        def _():
            process(p1, dbufB, elbB, erbB, rows1, ex1, gsemB, ssemB)
        return ()

    lax.fori_loop(0, (NSUB + 1) // 2, pair, ())
    drain(dbufA, elbA, erbA, rows0, ex0, ssemA)
    plsc.subcore_barrier()

    for k in range(NZCB):
        r0 = s * STRIPE + k * ZRB
        pltpu.sync_copy(U_sh.at[pl.ds(r0, ZRB)], obuf)
        pltpu.sync_copy(obuf, Up.at[c, pl.ds(r0, ZRB)])
    pltpu.sync_copy(den_sh.at[pl.ds(s * STRIPE, STRIPE)], zdbuf)
    pltpu.sync_copy(zdbuf, dp.at[c, 0, pl.ds(s * STRIPE, STRIPE)])
    plsc.subcore_barrier()


def _sc_gat_body(z0, el0, er0, sf0, df0, M0,
                 z1, el1, er1, sf1, df1, M1, zB,
                 U0p, d0p, U1p, d1p,
                 U_sh, den_sh, sfb, dfb, dbufA, dbufB,
                 elbA, elbB, erbA, erbB,
                 rows0, rows1, ex0, ex1, mbuf, zdbuf, zbuf, obuf,
                 gsemA, gsemB, ssemA, ssemB):
    c = lax.axis_index("c")
    s = lax.axis_index("s")
    wid = s * NC + c
    pltpu.sync_copy(zB, zbuf)
    for refs in ((z0, el0, er0, sf0, df0, M0, U0p, d0p),
                 (z1, el1, er1, sf1, df1, M1, U1p, d1p)):
        _sc_gat_edge(refs, U_sh, den_sh, sfb, dfb,
                     dbufA, dbufB, elbA, elbB, erbA, erbB,
                     rows0, rows1, ex0, ex1, mbuf, zdbuf, zbuf, obuf,
                     gsemA, gsemB, ssemA, ssemB, c, s, wid)


def _sc_gat(z0, el0, er0, sf0, df0, M0, z1, el1, er1, sf1, df1, M1, zB):
    k = pl.kernel(
        _sc_gat_body,
        out_type=[
            jax.ShapeDtypeStruct((NC, N_PAD, H), f32),
            jax.ShapeDtypeStruct((NC, 1, N_PAD), f32),
            jax.ShapeDtypeStruct((NC, N_PAD, H), f32),
            jax.ShapeDtypeStruct((NC, 1, N_PAD), f32),
        ],
        mesh=_sc_mesh,
        compiler_params=_sc_params_b,
        scratch_types=[
            pltpu.VMEM_SHARED((N_PAD, H), f32),
            pltpu.VMEM_SHARED((N_PAD,), f32),
            pltpu.VMEM((EPW,), i32),
            pltpu.VMEM((EPW,), i32),
            pltpu.VMEM((SUB,), i32),
            pltpu.VMEM((SUB,), i32),
            pltpu.VMEM((SUB,), f32),
            pltpu.VMEM((SUB,), f32),
            pltpu.VMEM((SUB,), f32),
            pltpu.VMEM((SUB,), f32),
            pltpu.VMEM((SUB, H), f32),
            pltpu.VMEM((SUB, H), f32),
            pltpu.VMEM((SUB,), f32),
            pltpu.VMEM((SUB,), f32),
            pltpu.VMEM((16,), f32),
            pltpu.VMEM((STRIPE,), f32),
            pltpu.VMEM((ZRB, H), f32),
            pltpu.VMEM((ZRB, H), f32),
            pltpu.SemaphoreType.DMA,
            pltpu.SemaphoreType.DMA,
            pltpu.SemaphoreType.DMA,
            pltpu.SemaphoreType.DMA,
        ],
    )
    return k(z0, el0, er0, sf0, df0, M0, z1, el1, er1, sf1, df1, M1, zB)


def kernel(x, edge_index_e0, timestamp_e0, edge_index_e1, timestamp_e1,
           time_w, time_b, Wp0, bp0, Wp1, bp1,
           Wg0, al0, ar0, bg0, Wg1, al1, ar1, bg1,
           W1, b1, W2, b2):
    src0 = edge_index_e0[0]
    dst0 = edge_index_e0[1]
    src1 = edge_index_e1[0]
    dst1 = edge_index_e1[1]
    ct = _current_time(timestamp_e0, timestamp_e1)
    T0, T1 = _time_encodings(timestamp_e0, timestamp_e1, ct, time_w, time_b)

    zA = jnp.zeros((ZR, TD), f32)
    S0p, c0p, S1p, c1p = _sc_time(T0, dst0, T1, dst1, zA)

    z0, z1, el0, er0, el1, er1, M0, M1 = _mid(
        S0p, c0p.reshape(NW, N).T, S1p, c1p.reshape(NW, N).T,
        x, Wp0, bp0, Wp1, bp1, Wg0, Wg1, al0, ar0, al1, ar1)

    zB = jnp.zeros((ZRB, H), f32)
    M0b = jnp.broadcast_to(M0.reshape(1), (16,))
    M1b = jnp.broadcast_to(M1.reshape(1), (16,))
    U0p, d0p, U1p, d1p = _sc_gat(
        z0, el0.reshape(N), er0.reshape(N), src0, dst0, M0b,
        z1, el1.reshape(N), er1.reshape(N), src1, dst1, M1b, zB)

    W2p = jnp.zeros((H, H), f32).at[:, :OUT].set(W2)
    b2p = jnp.zeros((H,), f32).at[:OUT].set(b2)
    logits_pad = _final(U0p, d0p.reshape(NC, N_PAD)[:, :N].T, U1p,
                        d1p.reshape(NC, N_PAD)[:, :N].T, bg0, bg1, W1, b1,
                        W2p, b2p)
    return logits_pad[:, :OUT]
